# Initial kernel scaffold; baseline (speedup 1.0000x reference)
import jax
import jax.numpy as jnp
from jax.experimental import pallas as pl


def kernel(x, edge_index, W1, b1, p1, W2, b2, p2, W3, b3, p3, lin1_W, lin1_b, lin2_W, lin2_b):
    raise NotImplementedError("write your pallas kernel here")


# speedup vs baseline: 17.2772x; 17.2772x over previous
import functools
import math

import jax
import jax.numpy as jnp
from jax import lax
from jax.experimental import pallas as pl
from jax.experimental.pallas import tpu as pltpu
from jax.experimental.pallas import tpu_sc as plsc

N = 100000
N_PAD = 100352
E = 1600000
E_PAD = 1638400
CH = 128
NSUB = 16
ROWS_PW = N_PAD // NSUB
RB = 2048
INT_MIN = -(2 ** 31)


def _sc_deg(a, srcp, dstp, za):
    mesh = plsc.VectorSubcoreMesh(core_axis_name="c", subcore_axis_name="s")

    @functools.partial(
        pl.kernel,
        out_type=jax.ShapeDtypeStruct((2 * N_PAD,), jnp.float32),
        mesh=mesh,
        scratch_types=[
            pltpu.VMEM((CH,), jnp.int32),
            pltpu.VMEM((CH,), jnp.int32),
            pltpu.VMEM((CH,), jnp.float32),
            pltpu.VMEM_SHARED((N_PAD,), jnp.float32),
            pltpu.SemaphoreType.DMA,
        ],
        compiler_params=pltpu.CompilerParams(use_tc_tiling_on_sc=False),
    )
    def body(a_hbm, src_hbm, dst_hbm, za_hbm, out_hbm, sidx, didx, avals, acc, sem):
        c = lax.axis_index("c")
        s = lax.axis_index("s")
        pltpu.sync_copy(za_hbm, acc.at[pl.ds(s * ROWS_PW, ROWS_PW)])
        plsc.subcore_barrier()
        w = c * NSUB + s
        n_chunks = E_PAD // 32 // CH

        def chunk(j, carry):
            base = w * (E_PAD // 32) + j * CH
            pltpu.sync_copy(src_hbm.at[pl.ds(base, CH)], sidx)
            pltpu.sync_copy(dst_hbm.at[pl.ds(base, CH)], didx)
            pltpu.async_copy(a_hbm.at[sidx], avals, sem).wait()
            pltpu.sync_copy(avals, acc.at[didx], add=True)
            return carry

        lax.fori_loop(0, n_chunks, chunk, 0)
        plsc.subcore_barrier()
        pltpu.sync_copy(acc.at[pl.ds(s * ROWS_PW, ROWS_PW)],
                        out_hbm.at[pl.ds(c * N_PAD + s * ROWS_PW, ROWS_PW)])

    return body(a, srcp, dstp, za)


def _sc_gather_add(y2, srcp, dstp, zb):
    mesh = plsc.VectorSubcoreMesh(core_axis_name="c", subcore_axis_name="s")

    @functools.partial(
        pl.kernel,
        out_type=jax.ShapeDtypeStruct((2 * N_PAD, 16), jnp.float32),
        mesh=mesh,
        scratch_types=[
            pltpu.VMEM((CH,), jnp.int32),
            pltpu.VMEM((CH,), jnp.int32),
            pltpu.VMEM((CH, 16), jnp.float32),
            pltpu.VMEM_SHARED((N_PAD, 16), jnp.float32),
            pltpu.SemaphoreType.DMA,
        ],
        compiler_params=pltpu.CompilerParams(use_tc_tiling_on_sc=False),
    )
    def body(y2_hbm, src_hbm, dst_hbm, zb_hbm, out_hbm, sidx, didx, rows, acc, sem):
        c = lax.axis_index("c")
        s = lax.axis_index("s")
        pltpu.sync_copy(zb_hbm, acc.at[pl.ds(s * ROWS_PW, ROWS_PW)])
        plsc.subcore_barrier()
        n_chunks = E_PAD // NSUB // CH
        off = c * N_PAD

        def chunk(j, carry):
            base = s * (E_PAD // NSUB) + j * CH
            pltpu.sync_copy(src_hbm.at[pl.ds(base, CH)], sidx)
            pltpu.sync_copy(dst_hbm.at[pl.ds(base, CH)], didx)
            for i in range(CH // 16):
                sl = pl.ds(i * 16, 16)
                sidx[sl] = sidx[sl] + off
            pltpu.async_copy(y2_hbm.at[sidx], rows, sem).wait()
            pltpu.sync_copy(rows, acc.at[didx], add=True)
            return carry

        lax.fori_loop(0, n_chunks, chunk, 0)
        plsc.subcore_barrier()
        pltpu.sync_copy(acc.at[pl.ds(s * ROWS_PW, ROWS_PW)],
                        out_hbm.at[pl.ds(c * N_PAD + s * ROWS_PW, ROWS_PW)])

    return body(y2, srcp, dstp, zb)


def _tc_matmul(s, h, W):
    F = h.shape[1]

    def body(s_ref, h_ref, w_ref, o_ref):
        o_ref[...] = jnp.dot(s_ref[...][:, None] * h_ref[...], w_ref[...],
                             preferred_element_type=jnp.float32)

    return pl.pallas_call(
        body,
        grid=(N_PAD // RB,),
        in_specs=[
            pl.BlockSpec((RB,), lambda i: (i,)),
            pl.BlockSpec((RB, F), lambda i: (i, 0)),
            pl.BlockSpec((F, 32), lambda i: (0, 0)),
        ],
        out_specs=pl.BlockSpec((RB, 32), lambda i: (i, 0)),
        out_shape=jax.ShapeDtypeStruct((N_PAD, 32), jnp.float32),
    )(s, h, W)


def _tc_prep(cdeg, a, xw):
    def body(cd_ref, a_ref, xw_ref, y2_ref, dis_ref):
        deg = cd_ref[0, :] + cd_ref[1, :]
        av = a_ref[...]
        dis = lax.rsqrt(av * deg + 1.0)
        g = av * dis
        xwv = xw_ref[...]
        y2_ref[0] = xwv[:, :16] * g[:, None]
        y2_ref[1] = xwv[:, 16:] * g[:, None]
        dis_ref[...] = dis

    return pl.pallas_call(
        body,
        grid=(N_PAD // RB,),
        in_specs=[
            pl.BlockSpec((2, RB), lambda i: (0, i)),
            pl.BlockSpec((RB,), lambda i: (i,)),
            pl.BlockSpec((RB, 32), lambda i: (i, 0)),
        ],
        out_specs=[
            pl.BlockSpec((2, RB, 16), lambda i: (0, i, 0)),
            pl.BlockSpec((RB,), lambda i: (i,)),
        ],
        out_shape=[
            jax.ShapeDtypeStruct((2, N_PAD, 16), jnp.float32),
            jax.ShapeDtypeStruct((N_PAD,), jnp.float32),
        ],
    )(cdeg, a, xw)


def _tc_post(acc2, xw, dis, a, b, p):
    def body(acc_ref, xw_ref, dis_ref, a_ref, b_ref, p_ref, h_ref, sc_ref, key_ref):
        dis = dis_ref[...]
        av = a_ref[...]
        acc = jnp.concatenate([acc_ref[0], acc_ref[1]], axis=1)
        pre = dis[:, None] * acc + (dis * dis)[:, None] * xw_ref[...] + b_ref[...][None, :]
        h = jnp.maximum(pre * av[:, None], 0.0)
        h_ref[...] = h
        pv = p_ref[...]
        pn = jnp.sqrt(jnp.sum(pv * pv)) + 1e-16
        proj = jnp.sum(h * (pv / pn)[None, :], axis=1)
        sc = jnp.tanh(proj)
        sc_ref[...] = sc
        bits = lax.bitcast_convert_type(sc, jnp.int32)
        key = bits ^ jnp.where(bits < 0, jnp.int32(0x7FFFFFFF), jnp.int32(0))
        key_ref[...] = jnp.where(av > 0, key, jnp.int32(INT_MIN))

    return pl.pallas_call(
        body,
        grid=(N_PAD // RB,),
        in_specs=[
            pl.BlockSpec((2, RB, 16), lambda i: (0, i, 0)),
            pl.BlockSpec((RB, 32), lambda i: (i, 0)),
            pl.BlockSpec((RB,), lambda i: (i,)),
            pl.BlockSpec((RB,), lambda i: (i,)),
            pl.BlockSpec((32,), lambda i: (0,)),
            pl.BlockSpec((32,), lambda i: (0,)),
        ],
        out_specs=[
            pl.BlockSpec((RB, 32), lambda i: (i, 0)),
            pl.BlockSpec((RB,), lambda i: (i,)),
            pl.BlockSpec((RB,), lambda i: (i,)),
        ],
        out_shape=[
            jax.ShapeDtypeStruct((N_PAD, 32), jnp.float32),
            jax.ShapeDtypeStruct((N_PAD,), jnp.float32),
            jax.ShapeDtypeStruct((N_PAD,), jnp.int32),
        ],
    )(acc2, xw, dis, a, b, p)


def _tc_search(keyw, k):
    def body(kw_ref, t_ref):
        kw = kw_ref[...]
        kk = jnp.int32(k)

        def bit_step(t, B):
            trial = B | (jnp.int32(1) << (31 - t))
            tcmp = trial ^ INT_MIN
            cnt = jnp.sum((kw >= tcmp).astype(jnp.int32))
            return jnp.where(cnt >= kk, trial, B)

        B = lax.fori_loop(0, 32, bit_step, jnp.int32(0))
        T = B ^ INT_MIN
        c_gt = jnp.sum((kw > T).astype(jnp.int32))
        need = kk - c_gt
        eq = kw == T
        idxw = (lax.broadcasted_iota(jnp.int32, (N_PAD // 128, 128), 0) * 128
                + lax.broadcasted_iota(jnp.int32, (N_PAD // 128, 128), 1))

        def i0_step(t, lohi):
            lo, hi = lohi
            mid = (lo + hi) // 2
            cnt = jnp.sum((eq & (idxw < mid)).astype(jnp.int32))
            take = cnt >= need
            return (jnp.where(take, lo, mid + 1), jnp.where(take, mid, hi))

        _, i0 = lax.fori_loop(0, 17, i0_step,
                              (jnp.int32(0), jnp.int32(N_PAD)))
        t_ref[...] = jnp.stack([T, i0])[None, :]

    return pl.pallas_call(
        body,
        in_specs=[pl.BlockSpec((N_PAD // 128, 128), lambda: (0, 0))],
        out_specs=pl.BlockSpec((1, 2), lambda: (0, 0)),
        out_shape=jax.ShapeDtypeStruct((1, 2), jnp.int32),
    )(keyw)


def _tc_transpose(h):
    def body(h_ref, o_ref):
        o_ref[...] = h_ref[...].T

    return pl.pallas_call(
        body,
        grid=(N_PAD // RB,),
        in_specs=[pl.BlockSpec((RB, 32), lambda i: (i, 0))],
        out_specs=pl.BlockSpec((32, RB), lambda i: (0, i)),
        out_shape=jax.ShapeDtypeStruct((32, N_PAD), jnp.float32),
    )(h)


def _tc_select(ti, key, score, hT, k):
    nb = N_PAD // RB

    def body(ti_ref, key_ref, sc_ref, ht_ref, s_ref, a_ref, mx_ref, sm_ref):
        i = pl.program_id(0)
        T = ti_ref[0, 0]
        i0 = ti_ref[0, 1]
        kv = key_ref[...]
        idx = i * RB + lax.broadcasted_iota(jnp.int32, (RB,), 0)
        sel = (kv > T) | ((kv == T) & (idx < i0))
        sv = jnp.where(sel, sc_ref[...], 0.0)
        s_ref[...] = sv
        a_ref[...] = sel.astype(jnp.float32)
        vals = sv[None, :] * ht_ref[...]
        bm = jnp.max(jnp.where(sel[None, :], vals, -jnp.inf), axis=1,
                     keepdims=True)
        bs = jnp.sum(vals, axis=1, keepdims=True)

        @pl.when(i == 0)
        def _():
            mx_ref[...] = jnp.full((32, 1), -jnp.inf, jnp.float32)
            sm_ref[...] = jnp.zeros((32, 1), jnp.float32)

        mx_ref[...] = jnp.maximum(mx_ref[...], bm)
        sm_ref[...] = sm_ref[...] + bs * (1.0 / k)

    return pl.pallas_call(
        body,
        grid=(nb,),
        in_specs=[
            pl.BlockSpec((1, 2), lambda i: (0, 0)),
            pl.BlockSpec((RB,), lambda i: (i,)),
            pl.BlockSpec((RB,), lambda i: (i,)),
            pl.BlockSpec((32, RB), lambda i: (0, i)),
        ],
        out_specs=[
            pl.BlockSpec((RB,), lambda i: (i,)),
            pl.BlockSpec((RB,), lambda i: (i,)),
            pl.BlockSpec((32, 1), lambda i: (0, 0)),
            pl.BlockSpec((32, 1), lambda i: (0, 0)),
        ],
        out_shape=[
            jax.ShapeDtypeStruct((N_PAD,), jnp.float32),
            jax.ShapeDtypeStruct((N_PAD,), jnp.float32),
            jax.ShapeDtypeStruct((32, 1), jnp.float32),
            jax.ShapeDtypeStruct((32, 1), jnp.float32),
        ],
    )(ti, key, score, hT)


def _tc_mlp(ros, lin1_W, lin1_b, lin2_W, lin2_b):
    def body(r1_ref, r2_ref, r3_ref, r4_ref, r5_ref, r6_ref,
             w1_ref, b1_ref, w2_ref, b2_ref, o_ref):
        z1 = b1_ref[...][None, :]
        for j, r in enumerate((r1_ref, r2_ref, r3_ref, r4_ref, r5_ref, r6_ref)):
            seg = w1_ref[pl.ds(32 * j, 32), :]
            z1 = z1 + jnp.sum(r[...] * seg, axis=0, keepdims=True)
        z1 = jnp.maximum(z1, 0.0)
        z2 = jnp.maximum(jnp.dot(z1, w2_ref[...],
                                 preferred_element_type=jnp.float32)
                         + b2_ref[...][None, :], 0.0)
        o_ref[...] = jnp.concatenate(
            [jnp.exp(z2[:, 0:3]), jax.nn.sigmoid(z2[:, 3:10])], axis=1)

    return pl.pallas_call(
        body,
        out_shape=jax.ShapeDtypeStruct((1, 10), jnp.float32),
    )(*ros, lin1_W, lin1_b, lin2_W, lin2_b)


def kernel(x, edge_index, W1, b1, p1, W2, b2, p2, W3, b3, p3,
           lin1_W, lin1_b, lin2_W, lin2_b):
    src = edge_index[0].astype(jnp.int32)
    dst = edge_index[1].astype(jnp.int32)
    srcp = jnp.concatenate([src, jnp.zeros((E_PAD - E,), jnp.int32)])
    dstp = jnp.concatenate([dst, jnp.full((E_PAD - E,), N, jnp.int32)])

    za = jnp.zeros((ROWS_PW,), jnp.float32)
    zb = jnp.zeros((ROWS_PW, 16), jnp.float32)

    h = jnp.concatenate([x, jnp.zeros((N_PAD - N, x.shape[1]), jnp.float32)])
    a = jnp.concatenate([jnp.ones((N,), jnp.float32),
                         jnp.zeros((N_PAD - N,), jnp.float32)])
    s = a

    n_alive = N
    readouts = []
    for (W, b, p) in ((W1, b1, p1), (W2, b2, p2), (W3, b3, p3)):
        xw = _tc_matmul(s, h, W)
        cdeg = _sc_deg(a, srcp, dstp, za).reshape(2, N_PAD)
        y2, dis = _tc_prep(cdeg, a, xw)
        acc2 = _sc_gather_add(y2.reshape(2 * N_PAD, 16), srcp, dstp, zb)
        acc2 = acc2.reshape(2, N_PAD, 16)
        h, score, key = _tc_post(acc2, xw, dis, a, b, p)
        k = int(math.ceil(0.6 * n_alive))
        ti = _tc_search(key.reshape(N_PAD // 128, 128), k)
        hT = _tc_transpose(h)
        s, a, mx, sm = _tc_select(ti, key, score, hT, k)
        readouts.extend([mx, sm])
        n_alive = k

    return _tc_mlp(readouts, lin1_W, lin1_b, lin2_W, lin2_b)


# speedup vs baseline: 17.8535x; 1.0334x over previous
import functools
import math

import jax
import jax.numpy as jnp
from jax import lax
from jax.experimental import pallas as pl
from jax.experimental.pallas import tpu as pltpu
from jax.experimental.pallas import tpu_sc as plsc

N = 100000
N_PAD = 100352
E = 1600000
E_PAD = 1638400
CH = 128
NSUB = 16
ROWS_PW = N_PAD // NSUB
RB = 2048
INT_MIN = -(2 ** 31)


def _sc_deg(a, srcp, dstp, za):
    mesh = plsc.VectorSubcoreMesh(core_axis_name="c", subcore_axis_name="s")

    @functools.partial(
        pl.kernel,
        out_type=jax.ShapeDtypeStruct((2 * N_PAD,), jnp.float32),
        mesh=mesh,
        scratch_types=[
            pltpu.VMEM((CH,), jnp.int32),
            pltpu.VMEM((CH,), jnp.int32),
            pltpu.VMEM((CH,), jnp.float32),
            pltpu.VMEM_SHARED((N_PAD,), jnp.float32),
            pltpu.SemaphoreType.DMA,
        ],
        compiler_params=pltpu.CompilerParams(use_tc_tiling_on_sc=False),
    )
    def body(a_hbm, src_hbm, dst_hbm, za_hbm, out_hbm, sidx, didx, avals, acc, sem):
        c = lax.axis_index("c")
        s = lax.axis_index("s")
        pltpu.sync_copy(za_hbm, acc.at[pl.ds(s * ROWS_PW, ROWS_PW)])
        plsc.subcore_barrier()
        w = c * NSUB + s
        n_chunks = E_PAD // 32 // CH

        def chunk(j, carry):
            base = w * (E_PAD // 32) + j * CH
            pltpu.sync_copy(src_hbm.at[pl.ds(base, CH)], sidx)
            pltpu.sync_copy(dst_hbm.at[pl.ds(base, CH)], didx)
            pltpu.async_copy(a_hbm.at[sidx], avals, sem).wait()
            pltpu.sync_copy(avals, acc.at[didx], add=True)
            return carry

        lax.fori_loop(0, n_chunks, chunk, 0)
        plsc.subcore_barrier()
        pltpu.sync_copy(acc.at[pl.ds(s * ROWS_PW, ROWS_PW)],
                        out_hbm.at[pl.ds(c * N_PAD + s * ROWS_PW, ROWS_PW)])

    return body(a, srcp, dstp, za)


def _sc_gather_add(y2, srcp, dstp, zb):
    mesh = plsc.VectorSubcoreMesh(core_axis_name="c", subcore_axis_name="s")

    @functools.partial(
        pl.kernel,
        out_type=jax.ShapeDtypeStruct((2 * N_PAD, 16), jnp.float32),
        mesh=mesh,
        scratch_types=[
            pltpu.VMEM((CH,), jnp.int32),
            pltpu.VMEM((CH,), jnp.int32),
            pltpu.VMEM((CH, 16), jnp.float32),
            pltpu.VMEM_SHARED((N_PAD, 16), jnp.float32),
            pltpu.SemaphoreType.DMA,
        ],
        compiler_params=pltpu.CompilerParams(use_tc_tiling_on_sc=False),
    )
    def body(y2_hbm, src_hbm, dst_hbm, zb_hbm, out_hbm, sidx, didx, rows, acc, sem):
        c = lax.axis_index("c")
        s = lax.axis_index("s")
        pltpu.sync_copy(zb_hbm, acc.at[pl.ds(s * ROWS_PW, ROWS_PW)])
        plsc.subcore_barrier()
        n_chunks = E_PAD // NSUB // CH
        off = c * N_PAD

        def chunk(j, carry):
            base = s * (E_PAD // NSUB) + j * CH
            pltpu.sync_copy(src_hbm.at[pl.ds(base, CH)], sidx)
            pltpu.sync_copy(dst_hbm.at[pl.ds(base, CH)], didx)
            for i in range(CH // 16):
                sl = pl.ds(i * 16, 16)
                sidx[sl] = sidx[sl] + off
            pltpu.async_copy(y2_hbm.at[sidx], rows, sem).wait()
            pltpu.sync_copy(rows, acc.at[didx], add=True)
            return carry

        lax.fori_loop(0, n_chunks, chunk, 0)
        plsc.subcore_barrier()
        pltpu.sync_copy(acc.at[pl.ds(s * ROWS_PW, ROWS_PW)],
                        out_hbm.at[pl.ds(c * N_PAD + s * ROWS_PW, ROWS_PW)])

    return body(y2, srcp, dstp, zb)


def _tc_matmul(s, h, W):
    F = h.shape[1]

    def body(s_ref, h_ref, w_ref, o_ref):
        o_ref[...] = jnp.dot(s_ref[...][:, None] * h_ref[...], w_ref[...],
                             preferred_element_type=jnp.float32)

    return pl.pallas_call(
        body,
        grid=(N_PAD // RB,),
        in_specs=[
            pl.BlockSpec((RB,), lambda i: (i,)),
            pl.BlockSpec((RB, F), lambda i: (i, 0)),
            pl.BlockSpec((F, 32), lambda i: (0, 0)),
        ],
        out_specs=pl.BlockSpec((RB, 32), lambda i: (i, 0)),
        out_shape=jax.ShapeDtypeStruct((N_PAD, 32), jnp.float32),
    )(s, h, W)


def _tc_prep(cdeg, a, xw):
    def body(cd_ref, a_ref, xw_ref, y2_ref, dis_ref):
        deg = cd_ref[0, :] + cd_ref[1, :]
        av = a_ref[...]
        dis = lax.rsqrt(av * deg + 1.0)
        g = av * dis
        xwv = xw_ref[...]
        y2_ref[0] = xwv[:, :16] * g[:, None]
        y2_ref[1] = xwv[:, 16:] * g[:, None]
        dis_ref[...] = dis

    return pl.pallas_call(
        body,
        grid=(N_PAD // RB,),
        in_specs=[
            pl.BlockSpec((2, RB), lambda i: (0, i)),
            pl.BlockSpec((RB,), lambda i: (i,)),
            pl.BlockSpec((RB, 32), lambda i: (i, 0)),
        ],
        out_specs=[
            pl.BlockSpec((2, RB, 16), lambda i: (0, i, 0)),
            pl.BlockSpec((RB,), lambda i: (i,)),
        ],
        out_shape=[
            jax.ShapeDtypeStruct((2, N_PAD, 16), jnp.float32),
            jax.ShapeDtypeStruct((N_PAD,), jnp.float32),
        ],
    )(cdeg, a, xw)


def _tc_post(acc2, xw, dis, a, b, p):
    def body(acc_ref, xw_ref, dis_ref, a_ref, b_ref, p_ref, h_ref, sc_ref, key_ref):
        dis = dis_ref[...]
        av = a_ref[...]
        acc = jnp.concatenate([acc_ref[0], acc_ref[1]], axis=1)
        pre = dis[:, None] * acc + (dis * dis)[:, None] * xw_ref[...] + b_ref[...][None, :]
        h = jnp.maximum(pre * av[:, None], 0.0)
        h_ref[...] = h
        pv = p_ref[...]
        pn = jnp.sqrt(jnp.sum(pv * pv)) + 1e-16
        proj = jnp.sum(h * (pv / pn)[None, :], axis=1)
        sc = jnp.tanh(proj)
        sc_ref[...] = sc
        bits = lax.bitcast_convert_type(sc, jnp.int32)
        key = bits ^ jnp.where(bits < 0, jnp.int32(0x7FFFFFFF), jnp.int32(0))
        key_ref[...] = jnp.where(av > 0, key, jnp.int32(INT_MIN))

    return pl.pallas_call(
        body,
        grid=(N_PAD // RB,),
        in_specs=[
            pl.BlockSpec((2, RB, 16), lambda i: (0, i, 0)),
            pl.BlockSpec((RB, 32), lambda i: (i, 0)),
            pl.BlockSpec((RB,), lambda i: (i,)),
            pl.BlockSpec((RB,), lambda i: (i,)),
            pl.BlockSpec((32,), lambda i: (0,)),
            pl.BlockSpec((32,), lambda i: (0,)),
        ],
        out_specs=[
            pl.BlockSpec((RB, 32), lambda i: (i, 0)),
            pl.BlockSpec((RB,), lambda i: (i,)),
            pl.BlockSpec((RB,), lambda i: (i,)),
        ],
        out_shape=[
            jax.ShapeDtypeStruct((N_PAD, 32), jnp.float32),
            jax.ShapeDtypeStruct((N_PAD,), jnp.float32),
            jax.ShapeDtypeStruct((N_PAD,), jnp.int32),
        ],
    )(acc2, xw, dis, a, b, p)


def _tc_search(keyw, k):
    def body(kw_ref, t_ref):
        kw = kw_ref[...]
        kk = jnp.int32(k)

        def bit_step(t, B):
            trial = B | (jnp.int32(1) << (31 - t))
            tcmp = trial ^ INT_MIN
            cnt = jnp.sum((kw >= tcmp).astype(jnp.int32))
            return jnp.where(cnt >= kk, trial, B)

        B = lax.fori_loop(0, 32, bit_step, jnp.int32(0))
        T = B ^ INT_MIN
        c_gt = jnp.sum((kw > T).astype(jnp.int32))
        need = kk - c_gt
        eq = kw == T
        idxw = (lax.broadcasted_iota(jnp.int32, (N_PAD // 128, 128), 0) * 128
                + lax.broadcasted_iota(jnp.int32, (N_PAD // 128, 128), 1))

        def i0_step(t, lohi):
            lo, hi = lohi
            mid = (lo + hi) // 2
            cnt = jnp.sum((eq & (idxw < mid)).astype(jnp.int32))
            take = cnt >= need
            return (jnp.where(take, lo, mid + 1), jnp.where(take, mid, hi))

        _, i0 = lax.fori_loop(0, 17, i0_step,
                              (jnp.int32(0), jnp.int32(N_PAD)))
        t_ref[...] = jnp.stack([T, i0])[None, :]

    return pl.pallas_call(
        body,
        in_specs=[pl.BlockSpec((N_PAD // 128, 128), lambda: (0, 0))],
        out_specs=pl.BlockSpec((1, 2), lambda: (0, 0)),
        out_shape=jax.ShapeDtypeStruct((1, 2), jnp.int32),
    )(keyw)


def _tc_transpose(h):
    def body(h_ref, o_ref):
        o_ref[...] = h_ref[...].T

    return pl.pallas_call(
        body,
        grid=(N_PAD // RB,),
        in_specs=[pl.BlockSpec((RB, 32), lambda i: (i, 0))],
        out_specs=pl.BlockSpec((32, RB), lambda i: (0, i)),
        out_shape=jax.ShapeDtypeStruct((32, N_PAD), jnp.float32),
    )(h)


def _tc_select(ti, key, score, hT, k):
    nb = N_PAD // RB

    def body(ti_ref, key_ref, sc_ref, ht_ref, s_ref, a_ref, mx_ref, sm_ref):
        i = pl.program_id(0)
        T = ti_ref[0, 0]
        i0 = ti_ref[0, 1]
        kv = key_ref[...]
        idx = i * RB + lax.broadcasted_iota(jnp.int32, (RB,), 0)
        sel = (kv > T) | ((kv == T) & (idx < i0))
        sv = jnp.where(sel, sc_ref[...], 0.0)
        s_ref[...] = sv
        a_ref[...] = sel.astype(jnp.float32)
        vals = sv[None, :] * ht_ref[...]
        bm = jnp.max(jnp.where(sel[None, :], vals, -jnp.inf), axis=1,
                     keepdims=True)
        bs = jnp.sum(vals, axis=1, keepdims=True)

        @pl.when(i == 0)
        def _():
            mx_ref[...] = jnp.full((32, 1), -jnp.inf, jnp.float32)
            sm_ref[...] = jnp.zeros((32, 1), jnp.float32)

        mx_ref[...] = jnp.maximum(mx_ref[...], bm)
        sm_ref[...] = sm_ref[...] + bs * (1.0 / k)

    return pl.pallas_call(
        body,
        grid=(nb,),
        in_specs=[
            pl.BlockSpec((1, 2), lambda i: (0, 0)),
            pl.BlockSpec((RB,), lambda i: (i,)),
            pl.BlockSpec((RB,), lambda i: (i,)),
            pl.BlockSpec((32, RB), lambda i: (0, i)),
        ],
        out_specs=[
            pl.BlockSpec((RB,), lambda i: (i,)),
            pl.BlockSpec((RB,), lambda i: (i,)),
            pl.BlockSpec((32, 1), lambda i: (0, 0)),
            pl.BlockSpec((32, 1), lambda i: (0, 0)),
        ],
        out_shape=[
            jax.ShapeDtypeStruct((N_PAD,), jnp.float32),
            jax.ShapeDtypeStruct((N_PAD,), jnp.float32),
            jax.ShapeDtypeStruct((32, 1), jnp.float32),
            jax.ShapeDtypeStruct((32, 1), jnp.float32),
        ],
    )(ti, key, score, hT)


def _tc_mlp(ros, lin1_W, lin1_b, lin2_W, lin2_b):
    def body(r1_ref, r2_ref, r3_ref, r4_ref, r5_ref, r6_ref,
             w1_ref, b1_ref, w2_ref, b2_ref, o_ref):
        z1 = b1_ref[...][None, :]
        for j, r in enumerate((r1_ref, r2_ref, r3_ref, r4_ref, r5_ref, r6_ref)):
            seg = w1_ref[pl.ds(32 * j, 32), :]
            z1 = z1 + jnp.sum(r[...] * seg, axis=0, keepdims=True)
        z1 = jnp.maximum(z1, 0.0)
        z2 = jnp.maximum(jnp.dot(z1, w2_ref[...],
                                 preferred_element_type=jnp.float32)
                         + b2_ref[...][None, :], 0.0)
        o_ref[...] = jnp.concatenate(
            [jnp.exp(z2[:, 0:3]), jax.nn.sigmoid(z2[:, 3:10])], axis=1)

    return pl.pallas_call(
        body,
        out_shape=jax.ShapeDtypeStruct((1, 10), jnp.float32),
    )(*ros, lin1_W, lin1_b, lin2_W, lin2_b)


def kernel(x, edge_index, W1, b1, p1, W2, b2, p2, W3, b3, p3,
           lin1_W, lin1_b, lin2_W, lin2_b):
    src = edge_index[0].astype(jnp.int32)
    dst = edge_index[1].astype(jnp.int32)
    pad = jnp.arange(E_PAD - E, dtype=jnp.int32)
    srcp = jnp.concatenate([src, pad % N])
    dstp = jnp.concatenate([dst, N + pad % (N_PAD - N)])

    za = jnp.zeros((ROWS_PW,), jnp.float32)
    zb = jnp.zeros((ROWS_PW, 16), jnp.float32)

    h = jnp.concatenate([x, jnp.zeros((N_PAD - N, x.shape[1]), jnp.float32)])
    a = jnp.concatenate([jnp.ones((N,), jnp.float32),
                         jnp.zeros((N_PAD - N,), jnp.float32)])
    s = a

    n_alive = N
    readouts = []
    for (W, b, p) in ((W1, b1, p1), (W2, b2, p2), (W3, b3, p3)):
        xw = _tc_matmul(s, h, W)
        cdeg = _sc_deg(a, srcp, dstp, za).reshape(2, N_PAD)
        y2, dis = _tc_prep(cdeg, a, xw)
        acc2 = _sc_gather_add(y2.reshape(2 * N_PAD, 16), srcp, dstp, zb)
        acc2 = acc2.reshape(2, N_PAD, 16)
        h, score, key = _tc_post(acc2, xw, dis, a, b, p)
        k = int(math.ceil(0.6 * n_alive))
        ti = _tc_search(key.reshape(N_PAD // 128, 128), k)
        hT = _tc_transpose(h)
        s, a, mx, sm = _tc_select(ti, key, score, hT, k)
        readouts.extend([mx, sm])
        n_alive = k

    return _tc_mlp(readouts, lin1_W, lin1_b, lin2_W, lin2_b)


# speedup vs baseline: 45.6498x; 2.5569x over previous
import functools
import math

import jax
import jax.numpy as jnp
from jax import lax
from jax.experimental import pallas as pl
from jax.experimental.pallas import tpu as pltpu
from jax.experimental.pallas import tpu_sc as plsc

N = 100000
N_PAD = 100352
E = 1600000
E_PAD = 1638400
CH = 1024
NSUB = 16
ROWS_PW = N_PAD // NSUB
RB = 2048
INT_MIN = -(2 ** 31)


def _sc_deg(a, srcp, dstp, za):
    mesh = plsc.VectorSubcoreMesh(core_axis_name="c", subcore_axis_name="s")

    @functools.partial(
        pl.kernel,
        out_type=jax.ShapeDtypeStruct((2 * N_PAD,), jnp.float32),
        mesh=mesh,
        scratch_types=[
            pltpu.VMEM((CH,), jnp.int32),
            pltpu.VMEM((CH,), jnp.int32),
            pltpu.VMEM((CH,), jnp.float32),
            pltpu.VMEM_SHARED((N_PAD,), jnp.float32),
            pltpu.SemaphoreType.DMA,
        ],
        compiler_params=pltpu.CompilerParams(use_tc_tiling_on_sc=False),
    )
    def body(a_hbm, src_hbm, dst_hbm, za_hbm, out_hbm, sidx, didx, avals, acc, sem):
        c = lax.axis_index("c")
        s = lax.axis_index("s")
        pltpu.sync_copy(za_hbm, acc.at[pl.ds(s * ROWS_PW, ROWS_PW)])
        plsc.subcore_barrier()
        w = c * NSUB + s
        n_chunks = E_PAD // 32 // CH

        def chunk(j, carry):
            base = w * (E_PAD // 32) + j * CH
            pltpu.sync_copy(src_hbm.at[pl.ds(base, CH)], sidx)
            pltpu.sync_copy(dst_hbm.at[pl.ds(base, CH)], didx)
            pltpu.async_copy(a_hbm.at[sidx], avals, sem).wait()
            pltpu.sync_copy(avals, acc.at[didx], add=True)
            return carry

        lax.fori_loop(0, n_chunks, chunk, 0)
        plsc.subcore_barrier()
        pltpu.sync_copy(acc.at[pl.ds(s * ROWS_PW, ROWS_PW)],
                        out_hbm.at[pl.ds(c * N_PAD + s * ROWS_PW, ROWS_PW)])

    return body(a, srcp, dstp, za)


def _sc_gather_add(y2, src2, dstp, zb):
    mesh = plsc.VectorSubcoreMesh(core_axis_name="c", subcore_axis_name="s")

    @functools.partial(
        pl.kernel,
        out_type=jax.ShapeDtypeStruct((2 * N_PAD, 16), jnp.float32),
        mesh=mesh,
        scratch_types=[
            pltpu.VMEM((CH,), jnp.int32),
            pltpu.VMEM((CH,), jnp.int32),
            pltpu.VMEM((CH, 16), jnp.float32),
            pltpu.VMEM_SHARED((N_PAD, 16), jnp.float32),
            pltpu.SemaphoreType.DMA,
        ],
        compiler_params=pltpu.CompilerParams(use_tc_tiling_on_sc=False),
    )
    def body(y2_hbm, src_hbm, dst_hbm, zb_hbm, out_hbm, sidx, didx, rows, acc, sem):
        c = lax.axis_index("c")
        s = lax.axis_index("s")
        pltpu.sync_copy(zb_hbm, acc.at[pl.ds(s * ROWS_PW, ROWS_PW)])
        plsc.subcore_barrier()
        n_chunks = E_PAD // NSUB // CH

        def chunk(j, carry):
            base = s * (E_PAD // NSUB) + j * CH
            pltpu.sync_copy(src_hbm.at[pl.ds(c * E_PAD + base, CH)], sidx)
            pltpu.sync_copy(dst_hbm.at[pl.ds(base, CH)], didx)
            pltpu.async_copy(y2_hbm.at[sidx], rows, sem).wait()
            pltpu.sync_copy(rows, acc.at[didx], add=True)
            return carry

        lax.fori_loop(0, n_chunks, chunk, 0)
        plsc.subcore_barrier()
        pltpu.sync_copy(acc.at[pl.ds(s * ROWS_PW, ROWS_PW)],
                        out_hbm.at[pl.ds(c * N_PAD + s * ROWS_PW, ROWS_PW)])

    return body(y2, src2, dstp, zb)


def _tc_matmul(s, h, W):
    F = h.shape[1]

    def body(s_ref, h_ref, w_ref, o_ref):
        o_ref[...] = jnp.dot(s_ref[...][:, None] * h_ref[...], w_ref[...],
                             preferred_element_type=jnp.float32)

    return pl.pallas_call(
        body,
        grid=(N_PAD // RB,),
        in_specs=[
            pl.BlockSpec((RB,), lambda i: (i,)),
            pl.BlockSpec((RB, F), lambda i: (i, 0)),
            pl.BlockSpec((F, 32), lambda i: (0, 0)),
        ],
        out_specs=pl.BlockSpec((RB, 32), lambda i: (i, 0)),
        out_shape=jax.ShapeDtypeStruct((N_PAD, 32), jnp.float32),
    )(s, h, W)


def _tc_prep(cdeg, a, xw):
    def body(cd_ref, a_ref, xw_ref, y2_ref, dis_ref):
        deg = cd_ref[0, :] + cd_ref[1, :]
        av = a_ref[...]
        dis = lax.rsqrt(av * deg + 1.0)
        g = av * dis
        xwv = xw_ref[...]
        y2_ref[0] = xwv[:, :16] * g[:, None]
        y2_ref[1] = xwv[:, 16:] * g[:, None]
        dis_ref[...] = dis

    return pl.pallas_call(
        body,
        grid=(N_PAD // RB,),
        in_specs=[
            pl.BlockSpec((2, RB), lambda i: (0, i)),
            pl.BlockSpec((RB,), lambda i: (i,)),
            pl.BlockSpec((RB, 32), lambda i: (i, 0)),
        ],
        out_specs=[
            pl.BlockSpec((2, RB, 16), lambda i: (0, i, 0)),
            pl.BlockSpec((RB,), lambda i: (i,)),
        ],
        out_shape=[
            jax.ShapeDtypeStruct((2, N_PAD, 16), jnp.float32),
            jax.ShapeDtypeStruct((N_PAD,), jnp.float32),
        ],
    )(cdeg, a, xw)


def _tc_post(acc2, xw, dis, a, b, p):
    def body(acc_ref, xw_ref, dis_ref, a_ref, b_ref, p_ref, h_ref, sc_ref, key_ref):
        dis = dis_ref[...]
        av = a_ref[...]
        acc = jnp.concatenate([acc_ref[0], acc_ref[1]], axis=1)
        pre = dis[:, None] * acc + (dis * dis)[:, None] * xw_ref[...] + b_ref[...][None, :]
        h = jnp.maximum(pre * av[:, None], 0.0)
        h_ref[...] = h
        pv = p_ref[...]
        pn = jnp.sqrt(jnp.sum(pv * pv)) + 1e-16
        proj = jnp.sum(h * (pv / pn)[None, :], axis=1)
        sc = jnp.tanh(proj)
        sc_ref[...] = sc
        bits = lax.bitcast_convert_type(sc, jnp.int32)
        key = bits ^ jnp.where(bits < 0, jnp.int32(0x7FFFFFFF), jnp.int32(0))
        key_ref[...] = jnp.where(av > 0, key, jnp.int32(INT_MIN))

    return pl.pallas_call(
        body,
        grid=(N_PAD // RB,),
        in_specs=[
            pl.BlockSpec((2, RB, 16), lambda i: (0, i, 0)),
            pl.BlockSpec((RB, 32), lambda i: (i, 0)),
            pl.BlockSpec((RB,), lambda i: (i,)),
            pl.BlockSpec((RB,), lambda i: (i,)),
            pl.BlockSpec((32,), lambda i: (0,)),
            pl.BlockSpec((32,), lambda i: (0,)),
        ],
        out_specs=[
            pl.BlockSpec((RB, 32), lambda i: (i, 0)),
            pl.BlockSpec((RB,), lambda i: (i,)),
            pl.BlockSpec((RB,), lambda i: (i,)),
        ],
        out_shape=[
            jax.ShapeDtypeStruct((N_PAD, 32), jnp.float32),
            jax.ShapeDtypeStruct((N_PAD,), jnp.float32),
            jax.ShapeDtypeStruct((N_PAD,), jnp.int32),
        ],
    )(acc2, xw, dis, a, b, p)


def _tc_search(keyw, k):
    def body(kw_ref, t_ref):
        kw = kw_ref[...]
        kk = jnp.int32(k)

        def bit_step(t, B):
            trial = B | (jnp.int32(1) << (31 - t))
            tcmp = trial ^ INT_MIN
            cnt = jnp.sum((kw >= tcmp).astype(jnp.int32))
            return jnp.where(cnt >= kk, trial, B)

        B = lax.fori_loop(0, 32, bit_step, jnp.int32(0))
        T = B ^ INT_MIN
        c_gt = jnp.sum((kw > T).astype(jnp.int32))
        need = kk - c_gt
        eq = kw == T
        idxw = (lax.broadcasted_iota(jnp.int32, (N_PAD // 128, 128), 0) * 128
                + lax.broadcasted_iota(jnp.int32, (N_PAD // 128, 128), 1))

        def i0_step(t, lohi):
            lo, hi = lohi
            mid = (lo + hi) // 2
            cnt = jnp.sum((eq & (idxw < mid)).astype(jnp.int32))
            take = cnt >= need
            return (jnp.where(take, lo, mid + 1), jnp.where(take, mid, hi))

        _, i0 = lax.fori_loop(0, 17, i0_step,
                              (jnp.int32(0), jnp.int32(N_PAD)))
        t_ref[...] = jnp.stack([T, i0])[None, :]

    return pl.pallas_call(
        body,
        in_specs=[pl.BlockSpec((N_PAD // 128, 128), lambda: (0, 0))],
        out_specs=pl.BlockSpec((1, 2), lambda: (0, 0)),
        out_shape=jax.ShapeDtypeStruct((1, 2), jnp.int32),
    )(keyw)


def _tc_transpose(h):
    def body(h_ref, o_ref):
        o_ref[...] = h_ref[...].T

    return pl.pallas_call(
        body,
        grid=(N_PAD // RB,),
        in_specs=[pl.BlockSpec((RB, 32), lambda i: (i, 0))],
        out_specs=pl.BlockSpec((32, RB), lambda i: (0, i)),
        out_shape=jax.ShapeDtypeStruct((32, N_PAD), jnp.float32),
    )(h)


def _tc_select(ti, key, score, hT, k):
    nb = N_PAD // RB

    def body(ti_ref, key_ref, sc_ref, ht_ref, s_ref, a_ref, mx_ref, sm_ref):
        i = pl.program_id(0)
        T = ti_ref[0, 0]
        i0 = ti_ref[0, 1]
        kv = key_ref[...]
        idx = i * RB + lax.broadcasted_iota(jnp.int32, (RB,), 0)
        sel = (kv > T) | ((kv == T) & (idx < i0))
        sv = jnp.where(sel, sc_ref[...], 0.0)
        s_ref[...] = sv
        a_ref[...] = sel.astype(jnp.float32)
        vals = sv[None, :] * ht_ref[...]
        bm = jnp.max(jnp.where(sel[None, :], vals, -jnp.inf), axis=1,
                     keepdims=True)
        bs = jnp.sum(vals, axis=1, keepdims=True)

        @pl.when(i == 0)
        def _():
            mx_ref[...] = jnp.full((32, 1), -jnp.inf, jnp.float32)
            sm_ref[...] = jnp.zeros((32, 1), jnp.float32)

        mx_ref[...] = jnp.maximum(mx_ref[...], bm)
        sm_ref[...] = sm_ref[...] + bs * (1.0 / k)

    return pl.pallas_call(
        body,
        grid=(nb,),
        in_specs=[
            pl.BlockSpec((1, 2), lambda i: (0, 0)),
            pl.BlockSpec((RB,), lambda i: (i,)),
            pl.BlockSpec((RB,), lambda i: (i,)),
            pl.BlockSpec((32, RB), lambda i: (0, i)),
        ],
        out_specs=[
            pl.BlockSpec((RB,), lambda i: (i,)),
            pl.BlockSpec((RB,), lambda i: (i,)),
            pl.BlockSpec((32, 1), lambda i: (0, 0)),
            pl.BlockSpec((32, 1), lambda i: (0, 0)),
        ],
        out_shape=[
            jax.ShapeDtypeStruct((N_PAD,), jnp.float32),
            jax.ShapeDtypeStruct((N_PAD,), jnp.float32),
            jax.ShapeDtypeStruct((32, 1), jnp.float32),
            jax.ShapeDtypeStruct((32, 1), jnp.float32),
        ],
    )(ti, key, score, hT)


def _tc_mlp(ros, lin1_W, lin1_b, lin2_W, lin2_b):
    def body(r1_ref, r2_ref, r3_ref, r4_ref, r5_ref, r6_ref,
             w1_ref, b1_ref, w2_ref, b2_ref, o_ref):
        z1 = b1_ref[...][None, :]
        for j, r in enumerate((r1_ref, r2_ref, r3_ref, r4_ref, r5_ref, r6_ref)):
            seg = w1_ref[pl.ds(32 * j, 32), :]
            z1 = z1 + jnp.sum(r[...] * seg, axis=0, keepdims=True)
        z1 = jnp.maximum(z1, 0.0)
        z2 = jnp.maximum(jnp.dot(z1, w2_ref[...],
                                 preferred_element_type=jnp.float32)
                         + b2_ref[...][None, :], 0.0)
        o_ref[...] = jnp.concatenate(
            [jnp.exp(z2[:, 0:3]), jax.nn.sigmoid(z2[:, 3:10])], axis=1)

    return pl.pallas_call(
        body,
        out_shape=jax.ShapeDtypeStruct((1, 10), jnp.float32),
    )(*ros, lin1_W, lin1_b, lin2_W, lin2_b)


def kernel(x, edge_index, W1, b1, p1, W2, b2, p2, W3, b3, p3,
           lin1_W, lin1_b, lin2_W, lin2_b):
    src = edge_index[0].astype(jnp.int32)
    dst = edge_index[1].astype(jnp.int32)
    pad = jnp.arange(E_PAD - E, dtype=jnp.int32)
    srcp = jnp.concatenate([src, pad % N])
    dstp = jnp.concatenate([dst, N + pad % (N_PAD - N)])
    src2 = jnp.concatenate([srcp, srcp + N_PAD])

    za = jnp.zeros((ROWS_PW,), jnp.float32)
    zb = jnp.zeros((ROWS_PW, 16), jnp.float32)

    h = jnp.concatenate([x, jnp.zeros((N_PAD - N, x.shape[1]), jnp.float32)])
    a = jnp.concatenate([jnp.ones((N,), jnp.float32),
                         jnp.zeros((N_PAD - N,), jnp.float32)])
    s = a

    n_alive = N
    readouts = []
    for (W, b, p) in ((W1, b1, p1), (W2, b2, p2), (W3, b3, p3)):
        xw = _tc_matmul(s, h, W)
        cdeg = _sc_deg(a, srcp, dstp, za).reshape(2, N_PAD)
        y2, dis = _tc_prep(cdeg, a, xw)
        acc2 = _sc_gather_add(y2.reshape(2 * N_PAD, 16), src2, dstp, zb)
        acc2 = acc2.reshape(2, N_PAD, 16)
        h, score, key = _tc_post(acc2, xw, dis, a, b, p)
        k = int(math.ceil(0.6 * n_alive))
        ti = _tc_search(key.reshape(N_PAD // 128, 128), k)
        hT = _tc_transpose(h)
        s, a, mx, sm = _tc_select(ti, key, score, hT, k)
        readouts.extend([mx, sm])
        n_alive = k

    return _tc_mlp(readouts, lin1_W, lin1_b, lin2_W, lin2_b)


# speedup vs baseline: 49.9963x; 1.0952x over previous
import functools
import math

import jax
import jax.numpy as jnp
from jax import lax
from jax.experimental import pallas as pl
from jax.experimental.pallas import tpu as pltpu
from jax.experimental.pallas import tpu_sc as plsc

N = 100000
N_PAD = 100352
E = 1600000
E_PAD = 1638400
CH = 1600
NSUB = 16
ROWS_PW = N_PAD // NSUB
RB = 2048
INT_MIN = -(2 ** 31)


def _sc_deg(a, srcp, dstp, za):
    mesh = plsc.VectorSubcoreMesh(core_axis_name="c", subcore_axis_name="s")

    @functools.partial(
        pl.kernel,
        out_type=jax.ShapeDtypeStruct((2 * N_PAD,), jnp.float32),
        mesh=mesh,
        scratch_types=[
            pltpu.VMEM((CH,), jnp.int32),
            pltpu.VMEM((CH,), jnp.int32),
            pltpu.VMEM((CH,), jnp.float32),
            pltpu.VMEM_SHARED((N_PAD,), jnp.float32),
            pltpu.SemaphoreType.DMA,
        ],
        compiler_params=pltpu.CompilerParams(use_tc_tiling_on_sc=False),
    )
    def body(a_hbm, src_hbm, dst_hbm, za_hbm, out_hbm, sidx, didx, avals, acc, sem):
        c = lax.axis_index("c")
        s = lax.axis_index("s")
        pltpu.sync_copy(za_hbm, acc.at[pl.ds(s * ROWS_PW, ROWS_PW)])
        plsc.subcore_barrier()
        w = c * NSUB + s
        n_chunks = E_PAD // 32 // CH

        def chunk(j, carry):
            base = w * (E_PAD // 32) + j * CH
            pltpu.sync_copy(src_hbm.at[pl.ds(base, CH)], sidx)
            pltpu.sync_copy(dst_hbm.at[pl.ds(base, CH)], didx)
            pltpu.async_copy(a_hbm.at[sidx], avals, sem).wait()
            pltpu.sync_copy(avals, acc.at[didx], add=True)
            return carry

        lax.fori_loop(0, n_chunks, chunk, 0)
        plsc.subcore_barrier()
        pltpu.sync_copy(acc.at[pl.ds(s * ROWS_PW, ROWS_PW)],
                        out_hbm.at[pl.ds(c * N_PAD + s * ROWS_PW, ROWS_PW)])

    return body(a, srcp, dstp, za)


def _sc_gather_add(y2, src2, dstp, zb):
    mesh = plsc.VectorSubcoreMesh(core_axis_name="c", subcore_axis_name="s")

    @functools.partial(
        pl.kernel,
        out_type=jax.ShapeDtypeStruct((2 * N_PAD, 16), jnp.float32),
        mesh=mesh,
        scratch_types=[
            pltpu.VMEM((CH,), jnp.int32),
            pltpu.VMEM((CH,), jnp.int32),
            pltpu.VMEM((CH, 16), jnp.float32),
            pltpu.VMEM_SHARED((N_PAD, 16), jnp.float32),
            pltpu.SemaphoreType.DMA,
        ],
        compiler_params=pltpu.CompilerParams(use_tc_tiling_on_sc=False),
    )
    def body(y2_hbm, src_hbm, dst_hbm, zb_hbm, out_hbm, sidx, didx, rows, acc, sem):
        c = lax.axis_index("c")
        s = lax.axis_index("s")
        pltpu.sync_copy(zb_hbm, acc.at[pl.ds(s * ROWS_PW, ROWS_PW)])
        plsc.subcore_barrier()
        n_chunks = E_PAD // NSUB // CH

        def chunk(j, carry):
            base = s * (E_PAD // NSUB) + j * CH
            pltpu.sync_copy(src_hbm.at[pl.ds(c * E_PAD + base, CH)], sidx)
            pltpu.sync_copy(dst_hbm.at[pl.ds(base, CH)], didx)
            pltpu.async_copy(y2_hbm.at[sidx], rows, sem).wait()
            pltpu.sync_copy(rows, acc.at[didx], add=True)
            return carry

        lax.fori_loop(0, n_chunks, chunk, 0)
        plsc.subcore_barrier()
        pltpu.sync_copy(acc.at[pl.ds(s * ROWS_PW, ROWS_PW)],
                        out_hbm.at[pl.ds(c * N_PAD + s * ROWS_PW, ROWS_PW)])

    return body(y2, src2, dstp, zb)


def _tc_matmul(s, h, W):
    F = h.shape[1]

    def body(s_ref, h_ref, w_ref, o_ref):
        o_ref[...] = jnp.dot(s_ref[...][:, None] * h_ref[...], w_ref[...],
                             preferred_element_type=jnp.float32)

    return pl.pallas_call(
        body,
        grid=(N_PAD // RB,),
        in_specs=[
            pl.BlockSpec((RB,), lambda i: (i,)),
            pl.BlockSpec((RB, F), lambda i: (i, 0)),
            pl.BlockSpec((F, 32), lambda i: (0, 0)),
        ],
        out_specs=pl.BlockSpec((RB, 32), lambda i: (i, 0)),
        out_shape=jax.ShapeDtypeStruct((N_PAD, 32), jnp.float32),
    )(s, h, W)


def _tc_prep(cdeg, a, xw):
    def body(cd_ref, a_ref, xw_ref, y2_ref, dis_ref):
        deg = cd_ref[0, :] + cd_ref[1, :]
        av = a_ref[...]
        dis = lax.rsqrt(av * deg + 1.0)
        g = av * dis
        xwv = xw_ref[...]
        y2_ref[0] = xwv[:, :16] * g[:, None]
        y2_ref[1] = xwv[:, 16:] * g[:, None]
        dis_ref[...] = dis

    return pl.pallas_call(
        body,
        grid=(N_PAD // RB,),
        in_specs=[
            pl.BlockSpec((2, RB), lambda i: (0, i)),
            pl.BlockSpec((RB,), lambda i: (i,)),
            pl.BlockSpec((RB, 32), lambda i: (i, 0)),
        ],
        out_specs=[
            pl.BlockSpec((2, RB, 16), lambda i: (0, i, 0)),
            pl.BlockSpec((RB,), lambda i: (i,)),
        ],
        out_shape=[
            jax.ShapeDtypeStruct((2, N_PAD, 16), jnp.float32),
            jax.ShapeDtypeStruct((N_PAD,), jnp.float32),
        ],
    )(cdeg, a, xw)


def _tc_post(acc2, xw, dis, a, b, p):
    def body(acc_ref, xw_ref, dis_ref, a_ref, b_ref, p_ref, h_ref, sc_ref, key_ref):
        dis = dis_ref[...]
        av = a_ref[...]
        acc = jnp.concatenate([acc_ref[0], acc_ref[1]], axis=1)
        pre = dis[:, None] * acc + (dis * dis)[:, None] * xw_ref[...] + b_ref[...][None, :]
        h = jnp.maximum(pre * av[:, None], 0.0)
        h_ref[...] = h
        pv = p_ref[...]
        pn = jnp.sqrt(jnp.sum(pv * pv)) + 1e-16
        proj = jnp.sum(h * (pv / pn)[None, :], axis=1)
        sc = jnp.tanh(proj)
        sc_ref[...] = sc
        bits = lax.bitcast_convert_type(sc, jnp.int32)
        key = bits ^ jnp.where(bits < 0, jnp.int32(0x7FFFFFFF), jnp.int32(0))
        key_ref[...] = jnp.where(av > 0, key, jnp.int32(INT_MIN))

    return pl.pallas_call(
        body,
        grid=(N_PAD // RB,),
        in_specs=[
            pl.BlockSpec((2, RB, 16), lambda i: (0, i, 0)),
            pl.BlockSpec((RB, 32), lambda i: (i, 0)),
            pl.BlockSpec((RB,), lambda i: (i,)),
            pl.BlockSpec((RB,), lambda i: (i,)),
            pl.BlockSpec((32,), lambda i: (0,)),
            pl.BlockSpec((32,), lambda i: (0,)),
        ],
        out_specs=[
            pl.BlockSpec((RB, 32), lambda i: (i, 0)),
            pl.BlockSpec((RB,), lambda i: (i,)),
            pl.BlockSpec((RB,), lambda i: (i,)),
        ],
        out_shape=[
            jax.ShapeDtypeStruct((N_PAD, 32), jnp.float32),
            jax.ShapeDtypeStruct((N_PAD,), jnp.float32),
            jax.ShapeDtypeStruct((N_PAD,), jnp.int32),
        ],
    )(acc2, xw, dis, a, b, p)


def _tc_search(keyw, k):
    def body(kw_ref, t_ref):
        kw = kw_ref[...]
        kk = jnp.int32(k)

        def bit_step(t, B):
            trial = B | (jnp.int32(1) << (31 - t))
            tcmp = trial ^ INT_MIN
            cnt = jnp.sum((kw >= tcmp).astype(jnp.int32))
            return jnp.where(cnt >= kk, trial, B)

        B = lax.fori_loop(0, 32, bit_step, jnp.int32(0))
        T = B ^ INT_MIN
        c_gt = jnp.sum((kw > T).astype(jnp.int32))
        need = kk - c_gt
        eq = kw == T
        idxw = (lax.broadcasted_iota(jnp.int32, (N_PAD // 128, 128), 0) * 128
                + lax.broadcasted_iota(jnp.int32, (N_PAD // 128, 128), 1))

        def i0_step(t, lohi):
            lo, hi = lohi
            mid = (lo + hi) // 2
            cnt = jnp.sum((eq & (idxw < mid)).astype(jnp.int32))
            take = cnt >= need
            return (jnp.where(take, lo, mid + 1), jnp.where(take, mid, hi))

        _, i0 = lax.fori_loop(0, 17, i0_step,
                              (jnp.int32(0), jnp.int32(N_PAD)))
        t_ref[...] = jnp.stack([T, i0])[None, :]

    return pl.pallas_call(
        body,
        in_specs=[pl.BlockSpec((N_PAD // 128, 128), lambda: (0, 0))],
        out_specs=pl.BlockSpec((1, 2), lambda: (0, 0)),
        out_shape=jax.ShapeDtypeStruct((1, 2), jnp.int32),
    )(keyw)


def _tc_transpose(h):
    def body(h_ref, o_ref):
        o_ref[...] = h_ref[...].T

    return pl.pallas_call(
        body,
        grid=(N_PAD // RB,),
        in_specs=[pl.BlockSpec((RB, 32), lambda i: (i, 0))],
        out_specs=pl.BlockSpec((32, RB), lambda i: (0, i)),
        out_shape=jax.ShapeDtypeStruct((32, N_PAD), jnp.float32),
    )(h)


def _tc_select(ti, key, score, hT, k):
    nb = N_PAD // RB

    def body(ti_ref, key_ref, sc_ref, ht_ref, s_ref, a_ref, mx_ref, sm_ref):
        i = pl.program_id(0)
        T = ti_ref[0, 0]
        i0 = ti_ref[0, 1]
        kv = key_ref[...]
        idx = i * RB + lax.broadcasted_iota(jnp.int32, (RB,), 0)
        sel = (kv > T) | ((kv == T) & (idx < i0))
        sv = jnp.where(sel, sc_ref[...], 0.0)
        s_ref[...] = sv
        a_ref[...] = sel.astype(jnp.float32)
        vals = sv[None, :] * ht_ref[...]
        bm = jnp.max(jnp.where(sel[None, :], vals, -jnp.inf), axis=1,
                     keepdims=True)
        bs = jnp.sum(vals, axis=1, keepdims=True)

        @pl.when(i == 0)
        def _():
            mx_ref[...] = jnp.full((32, 1), -jnp.inf, jnp.float32)
            sm_ref[...] = jnp.zeros((32, 1), jnp.float32)

        mx_ref[...] = jnp.maximum(mx_ref[...], bm)
        sm_ref[...] = sm_ref[...] + bs * (1.0 / k)

    return pl.pallas_call(
        body,
        grid=(nb,),
        in_specs=[
            pl.BlockSpec((1, 2), lambda i: (0, 0)),
            pl.BlockSpec((RB,), lambda i: (i,)),
            pl.BlockSpec((RB,), lambda i: (i,)),
            pl.BlockSpec((32, RB), lambda i: (0, i)),
        ],
        out_specs=[
            pl.BlockSpec((RB,), lambda i: (i,)),
            pl.BlockSpec((RB,), lambda i: (i,)),
            pl.BlockSpec((32, 1), lambda i: (0, 0)),
            pl.BlockSpec((32, 1), lambda i: (0, 0)),
        ],
        out_shape=[
            jax.ShapeDtypeStruct((N_PAD,), jnp.float32),
            jax.ShapeDtypeStruct((N_PAD,), jnp.float32),
            jax.ShapeDtypeStruct((32, 1), jnp.float32),
            jax.ShapeDtypeStruct((32, 1), jnp.float32),
        ],
    )(ti, key, score, hT)


def _tc_mlp(ros, lin1_W, lin1_b, lin2_W, lin2_b):
    def body(r1_ref, r2_ref, r3_ref, r4_ref, r5_ref, r6_ref,
             w1_ref, b1_ref, w2_ref, b2_ref, o_ref):
        z1 = b1_ref[...][None, :]
        for j, r in enumerate((r1_ref, r2_ref, r3_ref, r4_ref, r5_ref, r6_ref)):
            seg = w1_ref[pl.ds(32 * j, 32), :]
            z1 = z1 + jnp.sum(r[...] * seg, axis=0, keepdims=True)
        z1 = jnp.maximum(z1, 0.0)
        z2 = jnp.maximum(jnp.dot(z1, w2_ref[...],
                                 preferred_element_type=jnp.float32)
                         + b2_ref[...][None, :], 0.0)
        o_ref[...] = jnp.concatenate(
            [jnp.exp(z2[:, 0:3]), jax.nn.sigmoid(z2[:, 3:10])], axis=1)

    return pl.pallas_call(
        body,
        out_shape=jax.ShapeDtypeStruct((1, 10), jnp.float32),
    )(*ros, lin1_W, lin1_b, lin2_W, lin2_b)


def kernel(x, edge_index, W1, b1, p1, W2, b2, p2, W3, b3, p3,
           lin1_W, lin1_b, lin2_W, lin2_b):
    src = edge_index[0].astype(jnp.int32)
    dst = edge_index[1].astype(jnp.int32)
    pad = jnp.arange(E_PAD - E, dtype=jnp.int32)
    srcp = jnp.concatenate([src, pad % N])
    dstp = jnp.concatenate([dst, N + pad % (N_PAD - N)])
    src2 = jnp.concatenate([srcp, srcp + N_PAD])

    za = jnp.zeros((ROWS_PW,), jnp.float32)
    zb = jnp.zeros((ROWS_PW, 16), jnp.float32)

    h = jnp.concatenate([x, jnp.zeros((N_PAD - N, x.shape[1]), jnp.float32)])
    a = jnp.concatenate([jnp.ones((N,), jnp.float32),
                         jnp.zeros((N_PAD - N,), jnp.float32)])
    s = a

    n_alive = N
    readouts = []
    for (W, b, p) in ((W1, b1, p1), (W2, b2, p2), (W3, b3, p3)):
        xw = _tc_matmul(s, h, W)
        cdeg = _sc_deg(a, srcp, dstp, za).reshape(2, N_PAD)
        y2, dis = _tc_prep(cdeg, a, xw)
        acc2 = _sc_gather_add(y2.reshape(2 * N_PAD, 16), src2, dstp, zb)
        acc2 = acc2.reshape(2, N_PAD, 16)
        h, score, key = _tc_post(acc2, xw, dis, a, b, p)
        k = int(math.ceil(0.6 * n_alive))
        ti = _tc_search(key.reshape(N_PAD // 128, 128), k)
        hT = _tc_transpose(h)
        s, a, mx, sm = _tc_select(ti, key, score, hT, k)
        readouts.extend([mx, sm])
        n_alive = k

    return _tc_mlp(readouts, lin1_W, lin1_b, lin2_W, lin2_b)


# speedup vs baseline: 51.7785x; 1.0356x over previous
import functools
import math

import jax
import jax.numpy as jnp
from jax import lax
from jax.experimental import pallas as pl
from jax.experimental.pallas import tpu as pltpu
from jax.experimental.pallas import tpu_sc as plsc

N = 100000
N_PAD = 100352
E = 1600000
E_PAD = 1638400
CH = 1600
NSUB = 16
ROWS_PW = N_PAD // NSUB
RB = 2048
INT_MIN = -(2 ** 31)


def _sc_deg(a, srcp, dstp, za):
    mesh = plsc.VectorSubcoreMesh(core_axis_name="c", subcore_axis_name="s")

    @functools.partial(
        pl.kernel,
        out_type=jax.ShapeDtypeStruct((2 * N_PAD,), jnp.float32),
        mesh=mesh,
        scratch_types=[
            pltpu.VMEM((CH,), jnp.int32),
            pltpu.VMEM((CH,), jnp.int32),
            pltpu.VMEM((CH,), jnp.float32),
            pltpu.VMEM_SHARED((N_PAD,), jnp.float32),
            pltpu.SemaphoreType.DMA,
        ],
        compiler_params=pltpu.CompilerParams(use_tc_tiling_on_sc=False),
    )
    def body(a_hbm, src_hbm, dst_hbm, za_hbm, out_hbm, sidx, didx, avals, acc, sem):
        c = lax.axis_index("c")
        s = lax.axis_index("s")
        pltpu.sync_copy(za_hbm, acc.at[pl.ds(s * ROWS_PW, ROWS_PW)])
        plsc.subcore_barrier()
        w = c * NSUB + s
        n_chunks = E_PAD // 32 // CH

        def chunk(j, carry):
            base = w * (E_PAD // 32) + j * CH
            pltpu.sync_copy(src_hbm.at[pl.ds(base, CH)], sidx)
            pltpu.sync_copy(dst_hbm.at[pl.ds(base, CH)], didx)
            pltpu.async_copy(a_hbm.at[sidx], avals, sem).wait()
            pltpu.sync_copy(avals, acc.at[didx], add=True)
            return carry

        lax.fori_loop(0, n_chunks, chunk, 0)
        plsc.subcore_barrier()
        pltpu.sync_copy(acc.at[pl.ds(s * ROWS_PW, ROWS_PW)],
                        out_hbm.at[pl.ds(c * N_PAD + s * ROWS_PW, ROWS_PW)])

    return body(a, srcp, dstp, za)


def _sc_gather_add(y2, src2, dstp, zb):
    mesh = plsc.VectorSubcoreMesh(core_axis_name="c", subcore_axis_name="s")

    @functools.partial(
        pl.kernel,
        out_type=jax.ShapeDtypeStruct((2 * N_PAD, 16), jnp.float32),
        mesh=mesh,
        scratch_types=[
            pltpu.VMEM((CH,), jnp.int32),
            pltpu.VMEM((CH,), jnp.int32),
            pltpu.VMEM((CH, 16), jnp.float32),
            pltpu.VMEM_SHARED((N_PAD, 16), jnp.float32),
            pltpu.SemaphoreType.DMA,
        ],
        compiler_params=pltpu.CompilerParams(use_tc_tiling_on_sc=False),
    )
    def body(y2_hbm, src_hbm, dst_hbm, zb_hbm, out_hbm, sidx, didx, rows, acc, sem):
        c = lax.axis_index("c")
        s = lax.axis_index("s")
        pltpu.sync_copy(zb_hbm, acc.at[pl.ds(s * ROWS_PW, ROWS_PW)])
        plsc.subcore_barrier()
        n_chunks = E_PAD // NSUB // CH

        def chunk(j, carry):
            base = s * (E_PAD // NSUB) + j * CH
            pltpu.sync_copy(src_hbm.at[pl.ds(c * E_PAD + base, CH)], sidx)
            pltpu.sync_copy(dst_hbm.at[pl.ds(base, CH)], didx)
            pltpu.async_copy(y2_hbm.at[sidx], rows, sem).wait()
            pltpu.sync_copy(rows, acc.at[didx], add=True)
            return carry

        lax.fori_loop(0, n_chunks, chunk, 0)
        plsc.subcore_barrier()
        pltpu.sync_copy(acc.at[pl.ds(s * ROWS_PW, ROWS_PW)],
                        out_hbm.at[pl.ds(c * N_PAD + s * ROWS_PW, ROWS_PW)])

    return body(y2, src2, dstp, zb)


def _tc_mprep(cdeg, a, s, h, W):
    F = h.shape[1]

    def body(cd_ref, a_ref, s_ref, h_ref, w_ref, y2_ref, dis_ref, xw_ref):
        xw = jnp.dot(s_ref[...][:, None] * h_ref[...], w_ref[...],
                     preferred_element_type=jnp.float32)
        deg = cd_ref[0, :] + cd_ref[1, :]
        av = a_ref[...]
        dis = lax.rsqrt(av * deg + 1.0)
        g = av * dis
        y2_ref[0] = xw[:, :16] * g[:, None]
        y2_ref[1] = xw[:, 16:] * g[:, None]
        dis_ref[...] = dis
        xw_ref[...] = xw

    return pl.pallas_call(
        body,
        grid=(N_PAD // RB,),
        in_specs=[
            pl.BlockSpec((2, RB), lambda i: (0, i)),
            pl.BlockSpec((RB,), lambda i: (i,)),
            pl.BlockSpec((RB,), lambda i: (i,)),
            pl.BlockSpec((RB, F), lambda i: (i, 0)),
            pl.BlockSpec((F, 32), lambda i: (0, 0)),
        ],
        out_specs=[
            pl.BlockSpec((2, RB, 16), lambda i: (0, i, 0)),
            pl.BlockSpec((RB,), lambda i: (i,)),
            pl.BlockSpec((RB, 32), lambda i: (i, 0)),
        ],
        out_shape=[
            jax.ShapeDtypeStruct((2, N_PAD, 16), jnp.float32),
            jax.ShapeDtypeStruct((N_PAD,), jnp.float32),
            jax.ShapeDtypeStruct((N_PAD, 32), jnp.float32),
        ],
    )(cdeg, a, s, h, W)


def _tc_post(acc2, xw, dis, a, b, p):
    def body(acc_ref, xw_ref, dis_ref, a_ref, b_ref, p_ref, h_ref, ht_ref,
             sc_ref, key_ref):
        dis = dis_ref[...]
        av = a_ref[...]
        acc = jnp.concatenate([acc_ref[0], acc_ref[1]], axis=1)
        pre = dis[:, None] * acc + (dis * dis)[:, None] * xw_ref[...] + b_ref[...][None, :]
        h = jnp.maximum(pre * av[:, None], 0.0)
        h_ref[...] = h
        ht_ref[...] = h.T
        pv = p_ref[...]
        pn = jnp.sqrt(jnp.sum(pv * pv)) + 1e-16
        proj = jnp.sum(h * (pv / pn)[None, :], axis=1)
        sc = jnp.tanh(proj)
        sc_ref[...] = sc
        bits = lax.bitcast_convert_type(sc, jnp.int32)
        key = bits ^ jnp.where(bits < 0, jnp.int32(0x7FFFFFFF), jnp.int32(0))
        key_ref[...] = jnp.where(av > 0, key, jnp.int32(INT_MIN))

    return pl.pallas_call(
        body,
        grid=(N_PAD // RB,),
        in_specs=[
            pl.BlockSpec((2, RB, 16), lambda i: (0, i, 0)),
            pl.BlockSpec((RB, 32), lambda i: (i, 0)),
            pl.BlockSpec((RB,), lambda i: (i,)),
            pl.BlockSpec((RB,), lambda i: (i,)),
            pl.BlockSpec((32,), lambda i: (0,)),
            pl.BlockSpec((32,), lambda i: (0,)),
        ],
        out_specs=[
            pl.BlockSpec((RB, 32), lambda i: (i, 0)),
            pl.BlockSpec((32, RB), lambda i: (0, i)),
            pl.BlockSpec((RB,), lambda i: (i,)),
            pl.BlockSpec((RB,), lambda i: (i,)),
        ],
        out_shape=[
            jax.ShapeDtypeStruct((N_PAD, 32), jnp.float32),
            jax.ShapeDtypeStruct((32, N_PAD), jnp.float32),
            jax.ShapeDtypeStruct((N_PAD,), jnp.float32),
            jax.ShapeDtypeStruct((N_PAD,), jnp.int32),
        ],
    )(acc2, xw, dis, a, b, p)


def _tc_searchsel(keyw, key, score, hT, k):
    nb = N_PAD // RB

    def body(kw_ref, key_ref, sc_ref, ht_ref, s_ref, a_ref, mx_ref, sm_ref,
             ti_ref):
        i = pl.program_id(0)

        @pl.when(i == 0)
        def _():
            kw = kw_ref[...]
            kk = jnp.int32(k)

            def bit_step(t, B):
                trial = B | (jnp.int32(1) << (31 - t))
                tcmp = trial ^ INT_MIN
                cnt = jnp.sum((kw >= tcmp).astype(jnp.int32))
                return jnp.where(cnt >= kk, trial, B)

            B = lax.fori_loop(0, 32, bit_step, jnp.int32(0))
            T = B ^ INT_MIN
            c_gt = jnp.sum((kw > T).astype(jnp.int32))
            need = kk - c_gt
            eq = kw == T
            idxw = (lax.broadcasted_iota(jnp.int32, (N_PAD // 128, 128), 0) * 128
                    + lax.broadcasted_iota(jnp.int32, (N_PAD // 128, 128), 1))

            def i0_step(t, lohi):
                lo, hi = lohi
                mid = (lo + hi) // 2
                cnt = jnp.sum((eq & (idxw < mid)).astype(jnp.int32))
                take = cnt >= need
                return (jnp.where(take, lo, mid + 1), jnp.where(take, mid, hi))

            _, i0 = lax.fori_loop(0, 17, i0_step,
                                  (jnp.int32(0), jnp.int32(N_PAD)))
            ti_ref[0] = T
            ti_ref[1] = i0

        @pl.when(i > 0)
        def _():
            T = ti_ref[0]
            i0 = ti_ref[1]
            kv = key_ref[...]
            idx = (i - 1) * RB + lax.broadcasted_iota(jnp.int32, (RB,), 0)
            sel = (kv > T) | ((kv == T) & (idx < i0))
            sv = jnp.where(sel, sc_ref[...], 0.0)
            s_ref[...] = sv
            a_ref[...] = sel.astype(jnp.float32)
            vals = sv[None, :] * ht_ref[...]
            bm = jnp.max(jnp.where(sel[None, :], vals, -jnp.inf), axis=1,
                         keepdims=True)
            bs = jnp.sum(vals, axis=1, keepdims=True)

            @pl.when(i == 1)
            def _():
                mx_ref[...] = jnp.full((32, 1), -jnp.inf, jnp.float32)
                sm_ref[...] = jnp.zeros((32, 1), jnp.float32)

            mx_ref[...] = jnp.maximum(mx_ref[...], bm)
            sm_ref[...] = sm_ref[...] + bs * (1.0 / k)

    blk = lambda i: (jnp.maximum(i - 1, 0),)
    return pl.pallas_call(
        body,
        grid=(nb + 1,),
        in_specs=[
            pl.BlockSpec((N_PAD // 128, 128), lambda i: (0, 0)),
            pl.BlockSpec((RB,), blk),
            pl.BlockSpec((RB,), blk),
            pl.BlockSpec((32, RB), lambda i: (0, jnp.maximum(i - 1, 0))),
        ],
        out_specs=[
            pl.BlockSpec((RB,), blk),
            pl.BlockSpec((RB,), blk),
            pl.BlockSpec((32, 1), lambda i: (0, 0)),
            pl.BlockSpec((32, 1), lambda i: (0, 0)),
        ],
        out_shape=[
            jax.ShapeDtypeStruct((N_PAD,), jnp.float32),
            jax.ShapeDtypeStruct((N_PAD,), jnp.float32),
            jax.ShapeDtypeStruct((32, 1), jnp.float32),
            jax.ShapeDtypeStruct((32, 1), jnp.float32),
        ],
        scratch_shapes=[pltpu.SMEM((2,), jnp.int32)],
    )(keyw, key, score, hT)


def _tc_mlp(ros, lin1_W, lin1_b, lin2_W, lin2_b):
    def body(r1_ref, r2_ref, r3_ref, r4_ref, r5_ref, r6_ref,
             w1_ref, b1_ref, w2_ref, b2_ref, o_ref):
        z1 = b1_ref[...][None, :]
        for j, r in enumerate((r1_ref, r2_ref, r3_ref, r4_ref, r5_ref, r6_ref)):
            seg = w1_ref[pl.ds(32 * j, 32), :]
            z1 = z1 + jnp.sum(r[...] * seg, axis=0, keepdims=True)
        z1 = jnp.maximum(z1, 0.0)
        z2 = jnp.maximum(jnp.dot(z1, w2_ref[...],
                                 preferred_element_type=jnp.float32)
                         + b2_ref[...][None, :], 0.0)
        o_ref[...] = jnp.concatenate(
            [jnp.exp(z2[:, 0:3]), jax.nn.sigmoid(z2[:, 3:10])], axis=1)

    return pl.pallas_call(
        body,
        out_shape=jax.ShapeDtypeStruct((1, 10), jnp.float32),
    )(*ros, lin1_W, lin1_b, lin2_W, lin2_b)


def kernel(x, edge_index, W1, b1, p1, W2, b2, p2, W3, b3, p3,
           lin1_W, lin1_b, lin2_W, lin2_b):
    src = edge_index[0].astype(jnp.int32)
    dst = edge_index[1].astype(jnp.int32)
    pad = jnp.arange(E_PAD - E, dtype=jnp.int32)
    srcp = jnp.concatenate([src, pad % N])
    dstp = jnp.concatenate([dst, N + pad % (N_PAD - N)])
    src2 = jnp.concatenate([srcp, srcp + N_PAD])

    za = jnp.zeros((ROWS_PW,), jnp.float32)
    zb = jnp.zeros((ROWS_PW, 16), jnp.float32)

    h = jnp.concatenate([x, jnp.zeros((N_PAD - N, x.shape[1]), jnp.float32)])
    a = jnp.concatenate([jnp.ones((N,), jnp.float32),
                         jnp.zeros((N_PAD - N,), jnp.float32)])
    s = a

    n_alive = N
    readouts = []
    for (W, b, p) in ((W1, b1, p1), (W2, b2, p2), (W3, b3, p3)):
        cdeg = _sc_deg(a, srcp, dstp, za).reshape(2, N_PAD)
        y2, dis, xw = _tc_mprep(cdeg, a, s, h, W)
        acc2 = _sc_gather_add(y2.reshape(2 * N_PAD, 16), src2, dstp, zb)
        acc2 = acc2.reshape(2, N_PAD, 16)
        h, hT, score, key = _tc_post(acc2, xw, dis, a, b, p)
        k = int(math.ceil(0.6 * n_alive))
        s, a, mx, sm = _tc_searchsel(key.reshape(N_PAD // 128, 128), key,
                                     score, hT, k)
        readouts.extend([mx, sm])
        n_alive = k

    return _tc_mlp(readouts, lin1_W, lin1_b, lin2_W, lin2_b)


# speedup vs baseline: 58.6624x; 1.1329x over previous
import functools
import math

import jax
import jax.numpy as jnp
from jax import lax
from jax.experimental import pallas as pl
from jax.experimental.pallas import tpu as pltpu
from jax.experimental.pallas import tpu_sc as plsc

N = 100000
N_PAD = 100352
E = 1600000
E_PAD = 1638400
CH = 800
NSUB = 16
ROWS_PW = N_PAD // NSUB
RB = 2048
INT_MIN = -(2 ** 31)


def _sc_deg(a, srcp, dstp, za):
    mesh = plsc.VectorSubcoreMesh(core_axis_name="c", subcore_axis_name="s")

    @functools.partial(
        pl.kernel,
        out_type=jax.ShapeDtypeStruct((2 * N_PAD,), jnp.float32),
        mesh=mesh,
        scratch_types=[
            pltpu.VMEM((CH,), jnp.int32),
            pltpu.VMEM((CH,), jnp.int32),
            pltpu.VMEM((CH,), jnp.float32),
            pltpu.VMEM((CH,), jnp.int32),
            pltpu.VMEM((CH,), jnp.int32),
            pltpu.VMEM((CH,), jnp.float32),
            pltpu.VMEM_SHARED((N_PAD,), jnp.float32),
            pltpu.SemaphoreType.DMA,
            pltpu.SemaphoreType.DMA,
        ],
        compiler_params=pltpu.CompilerParams(use_tc_tiling_on_sc=False),
    )
    def body(a_hbm, src_hbm, dst_hbm, za_hbm, out_hbm,
             sidx0, didx0, av0, sidx1, didx1, av1, acc, sem0, sem1):
        c = lax.axis_index("c")
        s = lax.axis_index("s")
        pltpu.sync_copy(za_hbm, acc.at[pl.ds(s * ROWS_PW, ROWS_PW)])
        plsc.subcore_barrier()
        w = c * NSUB + s
        ebase = w * (E_PAD // 32)
        n_chunks = E_PAD // 32 // CH
        bufs = ((sidx0, didx0, av0, sem0), (sidx1, didx1, av1, sem1))

        for b in range(2):
            sidx, didx, av, sem = bufs[b]
            pltpu.sync_copy(src_hbm.at[pl.ds(ebase + b * CH, CH)], sidx)
            pltpu.sync_copy(dst_hbm.at[pl.ds(ebase + b * CH, CH)], didx)
            pltpu.async_copy(a_hbm.at[sidx], av, sem)

        def step(g, carry):
            for b in range(2):
                sidx, didx, av, sem = bufs[b]
                pltpu.make_async_copy(a_hbm.at[sidx], av, sem).wait()
                pltpu.sync_copy(av, acc.at[didx], add=True)
                base = ebase + (2 * g + b + 2) * CH
                pltpu.sync_copy(src_hbm.at[pl.ds(base, CH)], sidx)
                pltpu.sync_copy(dst_hbm.at[pl.ds(base, CH)], didx)
                pltpu.async_copy(a_hbm.at[sidx], av, sem)
            return carry

        lax.fori_loop(0, (n_chunks - 2) // 2, step, 0)
        for b in range(2):
            sidx, didx, av, sem = bufs[b]
            pltpu.make_async_copy(a_hbm.at[sidx], av, sem).wait()
            pltpu.sync_copy(av, acc.at[didx], add=True)
        plsc.subcore_barrier()
        pltpu.sync_copy(acc.at[pl.ds(s * ROWS_PW, ROWS_PW)],
                        out_hbm.at[pl.ds(c * N_PAD + s * ROWS_PW, ROWS_PW)])

    return body(a, srcp, dstp, za)


def _sc_gather_add(y2, src2, dstp, zb):
    mesh = plsc.VectorSubcoreMesh(core_axis_name="c", subcore_axis_name="s")

    @functools.partial(
        pl.kernel,
        out_type=jax.ShapeDtypeStruct((2 * N_PAD, 16), jnp.float32),
        mesh=mesh,
        scratch_types=[
            pltpu.VMEM((CH,), jnp.int32),
            pltpu.VMEM((CH,), jnp.int32),
            pltpu.VMEM((CH, 16), jnp.float32),
            pltpu.VMEM((CH,), jnp.int32),
            pltpu.VMEM((CH,), jnp.int32),
            pltpu.VMEM((CH, 16), jnp.float32),
            pltpu.VMEM_SHARED((N_PAD, 16), jnp.float32),
            pltpu.SemaphoreType.DMA,
            pltpu.SemaphoreType.DMA,
        ],
        compiler_params=pltpu.CompilerParams(use_tc_tiling_on_sc=False),
    )
    def body(y2_hbm, src_hbm, dst_hbm, zb_hbm, out_hbm,
             sidx0, didx0, rows0, sidx1, didx1, rows1, acc, sem0, sem1):
        c = lax.axis_index("c")
        s = lax.axis_index("s")
        pltpu.sync_copy(zb_hbm, acc.at[pl.ds(s * ROWS_PW, ROWS_PW)])
        plsc.subcore_barrier()
        ebase = s * (E_PAD // NSUB)
        n_chunks = E_PAD // NSUB // CH
        bufs = ((sidx0, didx0, rows0, sem0), (sidx1, didx1, rows1, sem1))

        for b in range(2):
            sidx, didx, rows, sem = bufs[b]
            pltpu.sync_copy(src_hbm.at[pl.ds(c * E_PAD + ebase + b * CH, CH)],
                            sidx)
            pltpu.sync_copy(dst_hbm.at[pl.ds(ebase + b * CH, CH)], didx)
            pltpu.async_copy(y2_hbm.at[sidx], rows, sem)

        def step(g, carry):
            for b in range(2):
                sidx, didx, rows, sem = bufs[b]
                pltpu.make_async_copy(y2_hbm.at[sidx], rows, sem).wait()
                pltpu.sync_copy(rows, acc.at[didx], add=True)
                base = ebase + (2 * g + b + 2) * CH
                pltpu.sync_copy(src_hbm.at[pl.ds(c * E_PAD + base, CH)], sidx)
                pltpu.sync_copy(dst_hbm.at[pl.ds(base, CH)], didx)
                pltpu.async_copy(y2_hbm.at[sidx], rows, sem)
            return carry

        lax.fori_loop(0, (n_chunks - 2) // 2, step, 0)
        for b in range(2):
            sidx, didx, rows, sem = bufs[b]
            pltpu.make_async_copy(y2_hbm.at[sidx], rows, sem).wait()
            pltpu.sync_copy(rows, acc.at[didx], add=True)
        plsc.subcore_barrier()
        pltpu.sync_copy(acc.at[pl.ds(s * ROWS_PW, ROWS_PW)],
                        out_hbm.at[pl.ds(c * N_PAD + s * ROWS_PW, ROWS_PW)])

    return body(y2, src2, dstp, zb)


def _tc_mprep(cdeg, a, s, h, W):
    F = h.shape[1]

    def body(cd_ref, a_ref, s_ref, h_ref, w_ref, y2_ref, dis_ref, xw_ref):
        xw = jnp.dot(s_ref[...][:, None] * h_ref[...], w_ref[...],
                     preferred_element_type=jnp.float32)
        deg = cd_ref[0, :] + cd_ref[1, :]
        av = a_ref[...]
        dis = lax.rsqrt(av * deg + 1.0)
        g = av * dis
        y2_ref[0] = xw[:, :16] * g[:, None]
        y2_ref[1] = xw[:, 16:] * g[:, None]
        dis_ref[...] = dis
        xw_ref[...] = xw

    return pl.pallas_call(
        body,
        grid=(N_PAD // RB,),
        in_specs=[
            pl.BlockSpec((2, RB), lambda i: (0, i)),
            pl.BlockSpec((RB,), lambda i: (i,)),
            pl.BlockSpec((RB,), lambda i: (i,)),
            pl.BlockSpec((RB, F), lambda i: (i, 0)),
            pl.BlockSpec((F, 32), lambda i: (0, 0)),
        ],
        out_specs=[
            pl.BlockSpec((2, RB, 16), lambda i: (0, i, 0)),
            pl.BlockSpec((RB,), lambda i: (i,)),
            pl.BlockSpec((RB, 32), lambda i: (i, 0)),
        ],
        out_shape=[
            jax.ShapeDtypeStruct((2, N_PAD, 16), jnp.float32),
            jax.ShapeDtypeStruct((N_PAD,), jnp.float32),
            jax.ShapeDtypeStruct((N_PAD, 32), jnp.float32),
        ],
    )(cdeg, a, s, h, W)


def _tc_post(acc2, xw, dis, a, b, p):
    def body(acc_ref, xw_ref, dis_ref, a_ref, b_ref, p_ref, h_ref, ht_ref,
             sc_ref, key_ref):
        dis = dis_ref[...]
        av = a_ref[...]
        acc = jnp.concatenate([acc_ref[0], acc_ref[1]], axis=1)
        pre = dis[:, None] * acc + (dis * dis)[:, None] * xw_ref[...] + b_ref[...][None, :]
        h = jnp.maximum(pre * av[:, None], 0.0)
        h_ref[...] = h
        ht_ref[...] = h.T
        pv = p_ref[...]
        pn = jnp.sqrt(jnp.sum(pv * pv)) + 1e-16
        proj = jnp.sum(h * (pv / pn)[None, :], axis=1)
        sc = jnp.tanh(proj)
        sc_ref[...] = sc
        bits = lax.bitcast_convert_type(sc, jnp.int32)
        key = bits ^ jnp.where(bits < 0, jnp.int32(0x7FFFFFFF), jnp.int32(0))
        key_ref[...] = jnp.where(av > 0, key, jnp.int32(INT_MIN))

    return pl.pallas_call(
        body,
        grid=(N_PAD // RB,),
        in_specs=[
            pl.BlockSpec((2, RB, 16), lambda i: (0, i, 0)),
            pl.BlockSpec((RB, 32), lambda i: (i, 0)),
            pl.BlockSpec((RB,), lambda i: (i,)),
            pl.BlockSpec((RB,), lambda i: (i,)),
            pl.BlockSpec((32,), lambda i: (0,)),
            pl.BlockSpec((32,), lambda i: (0,)),
        ],
        out_specs=[
            pl.BlockSpec((RB, 32), lambda i: (i, 0)),
            pl.BlockSpec((32, RB), lambda i: (0, i)),
            pl.BlockSpec((RB,), lambda i: (i,)),
            pl.BlockSpec((RB,), lambda i: (i,)),
        ],
        out_shape=[
            jax.ShapeDtypeStruct((N_PAD, 32), jnp.float32),
            jax.ShapeDtypeStruct((32, N_PAD), jnp.float32),
            jax.ShapeDtypeStruct((N_PAD,), jnp.float32),
            jax.ShapeDtypeStruct((N_PAD,), jnp.int32),
        ],
    )(acc2, xw, dis, a, b, p)


def _tc_searchsel(keyw, key, score, hT, k):
    nb = N_PAD // RB

    def body(kw_ref, key_ref, sc_ref, ht_ref, s_ref, a_ref, mx_ref, sm_ref,
             ti_ref):
        i = pl.program_id(0)

        @pl.when(i == 0)
        def _():
            kw = kw_ref[...]
            kk = jnp.int32(k)

            def bit_step(t, B):
                trial = B | (jnp.int32(1) << (31 - t))
                tcmp = trial ^ INT_MIN
                cnt = jnp.sum((kw >= tcmp).astype(jnp.int32))
                return jnp.where(cnt >= kk, trial, B)

            B = lax.fori_loop(0, 32, bit_step, jnp.int32(0))
            T = B ^ INT_MIN
            c_gt = jnp.sum((kw > T).astype(jnp.int32))
            need = kk - c_gt
            eq = kw == T
            idxw = (lax.broadcasted_iota(jnp.int32, (N_PAD // 128, 128), 0) * 128
                    + lax.broadcasted_iota(jnp.int32, (N_PAD // 128, 128), 1))

            def i0_step(t, lohi):
                lo, hi = lohi
                mid = (lo + hi) // 2
                cnt = jnp.sum((eq & (idxw < mid)).astype(jnp.int32))
                take = cnt >= need
                return (jnp.where(take, lo, mid + 1), jnp.where(take, mid, hi))

            _, i0 = lax.fori_loop(0, 17, i0_step,
                                  (jnp.int32(0), jnp.int32(N_PAD)))
            ti_ref[0] = T
            ti_ref[1] = i0

        @pl.when(i > 0)
        def _():
            T = ti_ref[0]
            i0 = ti_ref[1]
            kv = key_ref[...]
            idx = (i - 1) * RB + lax.broadcasted_iota(jnp.int32, (RB,), 0)
            sel = (kv > T) | ((kv == T) & (idx < i0))
            sv = jnp.where(sel, sc_ref[...], 0.0)
            s_ref[...] = sv
            a_ref[...] = sel.astype(jnp.float32)
            vals = sv[None, :] * ht_ref[...]
            bm = jnp.max(jnp.where(sel[None, :], vals, -jnp.inf), axis=1,
                         keepdims=True)
            bs = jnp.sum(vals, axis=1, keepdims=True)

            @pl.when(i == 1)
            def _():
                mx_ref[...] = jnp.full((32, 1), -jnp.inf, jnp.float32)
                sm_ref[...] = jnp.zeros((32, 1), jnp.float32)

            mx_ref[...] = jnp.maximum(mx_ref[...], bm)
            sm_ref[...] = sm_ref[...] + bs * (1.0 / k)

    blk = lambda i: (jnp.maximum(i - 1, 0),)
    return pl.pallas_call(
        body,
        grid=(nb + 1,),
        in_specs=[
            pl.BlockSpec((N_PAD // 128, 128), lambda i: (0, 0)),
            pl.BlockSpec((RB,), blk),
            pl.BlockSpec((RB,), blk),
            pl.BlockSpec((32, RB), lambda i: (0, jnp.maximum(i - 1, 0))),
        ],
        out_specs=[
            pl.BlockSpec((RB,), blk),
            pl.BlockSpec((RB,), blk),
            pl.BlockSpec((32, 1), lambda i: (0, 0)),
            pl.BlockSpec((32, 1), lambda i: (0, 0)),
        ],
        out_shape=[
            jax.ShapeDtypeStruct((N_PAD,), jnp.float32),
            jax.ShapeDtypeStruct((N_PAD,), jnp.float32),
            jax.ShapeDtypeStruct((32, 1), jnp.float32),
            jax.ShapeDtypeStruct((32, 1), jnp.float32),
        ],
        scratch_shapes=[pltpu.SMEM((2,), jnp.int32)],
    )(keyw, key, score, hT)


def _tc_mlp(ros, lin1_W, lin1_b, lin2_W, lin2_b):
    def body(r1_ref, r2_ref, r3_ref, r4_ref, r5_ref, r6_ref,
             w1_ref, b1_ref, w2_ref, b2_ref, o_ref):
        z1 = b1_ref[...][None, :]
        for j, r in enumerate((r1_ref, r2_ref, r3_ref, r4_ref, r5_ref, r6_ref)):
            seg = w1_ref[pl.ds(32 * j, 32), :]
            z1 = z1 + jnp.sum(r[...] * seg, axis=0, keepdims=True)
        z1 = jnp.maximum(z1, 0.0)
        z2 = jnp.maximum(jnp.dot(z1, w2_ref[...],
                                 preferred_element_type=jnp.float32)
                         + b2_ref[...][None, :], 0.0)
        o_ref[...] = jnp.concatenate(
            [jnp.exp(z2[:, 0:3]), jax.nn.sigmoid(z2[:, 3:10])], axis=1)

    return pl.pallas_call(
        body,
        out_shape=jax.ShapeDtypeStruct((1, 10), jnp.float32),
    )(*ros, lin1_W, lin1_b, lin2_W, lin2_b)


def kernel(x, edge_index, W1, b1, p1, W2, b2, p2, W3, b3, p3,
           lin1_W, lin1_b, lin2_W, lin2_b):
    src = edge_index[0].astype(jnp.int32)
    dst = edge_index[1].astype(jnp.int32)
    pad = jnp.arange(E_PAD - E, dtype=jnp.int32)
    srcp = jnp.concatenate([src, pad % N])
    dstp = jnp.concatenate([dst, N + pad % (N_PAD - N)])
    src2 = jnp.concatenate([srcp, srcp + N_PAD])

    za = jnp.zeros((ROWS_PW,), jnp.float32)
    zb = jnp.zeros((ROWS_PW, 16), jnp.float32)

    h = jnp.concatenate([x, jnp.zeros((N_PAD - N, x.shape[1]), jnp.float32)])
    a = jnp.concatenate([jnp.ones((N,), jnp.float32),
                         jnp.zeros((N_PAD - N,), jnp.float32)])
    s = a

    n_alive = N
    readouts = []
    for (W, b, p) in ((W1, b1, p1), (W2, b2, p2), (W3, b3, p3)):
        cdeg = _sc_deg(a, srcp, dstp, za).reshape(2, N_PAD)
        y2, dis, xw = _tc_mprep(cdeg, a, s, h, W)
        acc2 = _sc_gather_add(y2.reshape(2 * N_PAD, 16), src2, dstp, zb)
        acc2 = acc2.reshape(2, N_PAD, 16)
        h, hT, score, key = _tc_post(acc2, xw, dis, a, b, p)
        k = int(math.ceil(0.6 * n_alive))
        s, a, mx, sm = _tc_searchsel(key.reshape(N_PAD // 128, 128), key,
                                     score, hT, k)
        readouts.extend([mx, sm])
        n_alive = k

    return _tc_mlp(readouts, lin1_W, lin1_b, lin2_W, lin2_b)


# speedup vs baseline: 60.2567x; 1.0272x over previous
import functools
import math

import jax
import jax.numpy as jnp
from jax import lax
from jax.experimental import pallas as pl
from jax.experimental.pallas import tpu as pltpu
from jax.experimental.pallas import tpu_sc as plsc

N = 100000
N_PAD = 100352
E = 1600000
E_PAD = 1638400
CH = 800
CHA = 3200
NSUB = 16
ROWS_PW = N_PAD // NSUB
RB = 2048
INT_MIN = -(2 ** 31)


def _sc_deg(a, srcp, dstp, za):
    mesh = plsc.VectorSubcoreMesh(core_axis_name="c", subcore_axis_name="s")

    @functools.partial(
        pl.kernel,
        out_type=jax.ShapeDtypeStruct((2 * N_PAD,), jnp.float32),
        mesh=mesh,
        scratch_types=[
            pltpu.VMEM((CHA,), jnp.int32),
            pltpu.VMEM((CHA,), jnp.int32),
            pltpu.VMEM((CHA,), jnp.float32),
            pltpu.VMEM((CHA,), jnp.int32),
            pltpu.VMEM((CHA,), jnp.int32),
            pltpu.VMEM((CHA,), jnp.float32),
            pltpu.VMEM_SHARED((N_PAD,), jnp.float32),
            pltpu.SemaphoreType.DMA,
            pltpu.SemaphoreType.DMA,
        ],
        compiler_params=pltpu.CompilerParams(use_tc_tiling_on_sc=False),
    )
    def body(a_hbm, src_hbm, dst_hbm, za_hbm, out_hbm,
             sidx0, didx0, av0, sidx1, didx1, av1, acc, sem0, sem1):
        c = lax.axis_index("c")
        s = lax.axis_index("s")
        pltpu.sync_copy(za_hbm, acc.at[pl.ds(s * ROWS_PW, ROWS_PW)])
        plsc.subcore_barrier()
        w = c * NSUB + s
        ebase = w * (E_PAD // 32)
        n_chunks = E_PAD // 32 // CHA
        bufs = ((sidx0, didx0, av0, sem0), (sidx1, didx1, av1, sem1))

        for b in range(2):
            sidx, didx, av, sem = bufs[b]
            pltpu.sync_copy(src_hbm.at[pl.ds(ebase + b * CHA, CHA)], sidx)
            pltpu.sync_copy(dst_hbm.at[pl.ds(ebase + b * CHA, CHA)], didx)
            pltpu.async_copy(a_hbm.at[sidx], av, sem)

        def step(g, carry):
            for b in range(2):
                sidx, didx, av, sem = bufs[b]
                pltpu.make_async_copy(a_hbm.at[sidx], av, sem).wait()
                pltpu.sync_copy(av, acc.at[didx], add=True)
                base = ebase + (2 * g + b + 2) * CHA
                pltpu.sync_copy(src_hbm.at[pl.ds(base, CHA)], sidx)
                pltpu.sync_copy(dst_hbm.at[pl.ds(base, CHA)], didx)
                pltpu.async_copy(a_hbm.at[sidx], av, sem)
            return carry

        lax.fori_loop(0, (n_chunks - 2) // 2, step, 0)
        for b in range(2):
            sidx, didx, av, sem = bufs[b]
            pltpu.make_async_copy(a_hbm.at[sidx], av, sem).wait()
            pltpu.sync_copy(av, acc.at[didx], add=True)
        plsc.subcore_barrier()
        pltpu.sync_copy(acc.at[pl.ds(s * ROWS_PW, ROWS_PW)],
                        out_hbm.at[pl.ds(c * N_PAD + s * ROWS_PW, ROWS_PW)])

    return body(a, srcp, dstp, za)


def _sc_gather_add(y2, src2, dstp, zb):
    mesh = plsc.VectorSubcoreMesh(core_axis_name="c", subcore_axis_name="s")

    @functools.partial(
        pl.kernel,
        out_type=jax.ShapeDtypeStruct((2 * N_PAD, 16), jnp.float32),
        mesh=mesh,
        scratch_types=[
            pltpu.VMEM((CH,), jnp.int32),
            pltpu.VMEM((CH,), jnp.int32),
            pltpu.VMEM((CH, 16), jnp.float32),
            pltpu.VMEM((CH,), jnp.int32),
            pltpu.VMEM((CH,), jnp.int32),
            pltpu.VMEM((CH, 16), jnp.float32),
            pltpu.VMEM_SHARED((N_PAD, 16), jnp.float32),
            pltpu.SemaphoreType.DMA,
            pltpu.SemaphoreType.DMA,
        ],
        compiler_params=pltpu.CompilerParams(use_tc_tiling_on_sc=False),
    )
    def body(y2_hbm, src_hbm, dst_hbm, zb_hbm, out_hbm,
             sidx0, didx0, rows0, sidx1, didx1, rows1, acc, sem0, sem1):
        c = lax.axis_index("c")
        s = lax.axis_index("s")
        pltpu.sync_copy(zb_hbm, acc.at[pl.ds(s * ROWS_PW, ROWS_PW)])
        plsc.subcore_barrier()
        ebase = s * (E_PAD // NSUB)
        n_chunks = E_PAD // NSUB // CH
        bufs = ((sidx0, didx0, rows0, sem0), (sidx1, didx1, rows1, sem1))

        for b in range(2):
            sidx, didx, rows, sem = bufs[b]
            pltpu.sync_copy(src_hbm.at[pl.ds(c * E_PAD + ebase + b * CH, CH)],
                            sidx)
            pltpu.sync_copy(dst_hbm.at[pl.ds(ebase + b * CH, CH)], didx)
            pltpu.async_copy(y2_hbm.at[sidx], rows, sem)

        def step(g, carry):
            for b in range(2):
                sidx, didx, rows, sem = bufs[b]
                pltpu.make_async_copy(y2_hbm.at[sidx], rows, sem).wait()
                pltpu.sync_copy(rows, acc.at[didx], add=True)
                base = ebase + (2 * g + b + 2) * CH
                pltpu.sync_copy(src_hbm.at[pl.ds(c * E_PAD + base, CH)], sidx)
                pltpu.sync_copy(dst_hbm.at[pl.ds(base, CH)], didx)
                pltpu.async_copy(y2_hbm.at[sidx], rows, sem)
            return carry

        lax.fori_loop(0, (n_chunks - 2) // 2, step, 0)
        for b in range(2):
            sidx, didx, rows, sem = bufs[b]
            pltpu.make_async_copy(y2_hbm.at[sidx], rows, sem).wait()
            pltpu.sync_copy(rows, acc.at[didx], add=True)
        plsc.subcore_barrier()
        pltpu.sync_copy(acc.at[pl.ds(s * ROWS_PW, ROWS_PW)],
                        out_hbm.at[pl.ds(c * N_PAD + s * ROWS_PW, ROWS_PW)])

    return body(y2, src2, dstp, zb)


def _tc_mprep(cdeg, a, s, h, W):
    F = h.shape[1]

    def body(cd_ref, a_ref, s_ref, h_ref, w_ref, y2_ref, dis_ref, xw_ref):
        xw = jnp.dot(s_ref[...][:, None] * h_ref[...], w_ref[...],
                     preferred_element_type=jnp.float32)
        deg = cd_ref[0, :] + cd_ref[1, :]
        av = a_ref[...]
        dis = lax.rsqrt(av * deg + 1.0)
        g = av * dis
        y2_ref[0] = xw[:, :16] * g[:, None]
        y2_ref[1] = xw[:, 16:] * g[:, None]
        dis_ref[...] = dis
        xw_ref[...] = xw

    return pl.pallas_call(
        body,
        grid=(N_PAD // RB,),
        in_specs=[
            pl.BlockSpec((2, RB), lambda i: (0, i)),
            pl.BlockSpec((RB,), lambda i: (i,)),
            pl.BlockSpec((RB,), lambda i: (i,)),
            pl.BlockSpec((RB, F), lambda i: (i, 0)),
            pl.BlockSpec((F, 32), lambda i: (0, 0)),
        ],
        out_specs=[
            pl.BlockSpec((2, RB, 16), lambda i: (0, i, 0)),
            pl.BlockSpec((RB,), lambda i: (i,)),
            pl.BlockSpec((RB, 32), lambda i: (i, 0)),
        ],
        out_shape=[
            jax.ShapeDtypeStruct((2, N_PAD, 16), jnp.float32),
            jax.ShapeDtypeStruct((N_PAD,), jnp.float32),
            jax.ShapeDtypeStruct((N_PAD, 32), jnp.float32),
        ],
    )(cdeg, a, s, h, W)


def _tc_post(acc2, xw, dis, a, b, p):
    def body(acc_ref, xw_ref, dis_ref, a_ref, b_ref, p_ref, h_ref, ht_ref,
             sc_ref, key_ref):
        dis = dis_ref[...]
        av = a_ref[...]
        acc = jnp.concatenate([acc_ref[0], acc_ref[1]], axis=1)
        pre = dis[:, None] * acc + (dis * dis)[:, None] * xw_ref[...] + b_ref[...][None, :]
        h = jnp.maximum(pre * av[:, None], 0.0)
        h_ref[...] = h
        ht_ref[...] = h.T
        pv = p_ref[...]
        pn = jnp.sqrt(jnp.sum(pv * pv)) + 1e-16
        proj = jnp.sum(h * (pv / pn)[None, :], axis=1)
        sc = jnp.tanh(proj)
        sc_ref[...] = sc
        bits = lax.bitcast_convert_type(sc, jnp.int32)
        key = bits ^ jnp.where(bits < 0, jnp.int32(0x7FFFFFFF), jnp.int32(0))
        key_ref[...] = jnp.where(av > 0, key, jnp.int32(INT_MIN))

    return pl.pallas_call(
        body,
        grid=(N_PAD // RB,),
        in_specs=[
            pl.BlockSpec((2, RB, 16), lambda i: (0, i, 0)),
            pl.BlockSpec((RB, 32), lambda i: (i, 0)),
            pl.BlockSpec((RB,), lambda i: (i,)),
            pl.BlockSpec((RB,), lambda i: (i,)),
            pl.BlockSpec((32,), lambda i: (0,)),
            pl.BlockSpec((32,), lambda i: (0,)),
        ],
        out_specs=[
            pl.BlockSpec((RB, 32), lambda i: (i, 0)),
            pl.BlockSpec((32, RB), lambda i: (0, i)),
            pl.BlockSpec((RB,), lambda i: (i,)),
            pl.BlockSpec((RB,), lambda i: (i,)),
        ],
        out_shape=[
            jax.ShapeDtypeStruct((N_PAD, 32), jnp.float32),
            jax.ShapeDtypeStruct((32, N_PAD), jnp.float32),
            jax.ShapeDtypeStruct((N_PAD,), jnp.float32),
            jax.ShapeDtypeStruct((N_PAD,), jnp.int32),
        ],
    )(acc2, xw, dis, a, b, p)


def _tc_searchsel(keyw, key, score, hT, k):
    nb = N_PAD // RB

    def body(kw_ref, key_ref, sc_ref, ht_ref, s_ref, a_ref, mx_ref, sm_ref,
             ti_ref):
        i = pl.program_id(0)

        @pl.when(i == 0)
        def _():
            kw = kw_ref[...]
            kk = jnp.int32(k)

            def bit_step(t, B):
                trial = B | (jnp.int32(1) << (31 - t))
                tcmp = trial ^ INT_MIN
                cnt = jnp.sum((kw >= tcmp).astype(jnp.int32))
                return jnp.where(cnt >= kk, trial, B)

            B = lax.fori_loop(0, 32, bit_step, jnp.int32(0))
            T = B ^ INT_MIN
            c_gt = jnp.sum((kw > T).astype(jnp.int32))
            need = kk - c_gt
            eq = kw == T
            idxw = (lax.broadcasted_iota(jnp.int32, (N_PAD // 128, 128), 0) * 128
                    + lax.broadcasted_iota(jnp.int32, (N_PAD // 128, 128), 1))

            def i0_step(t, lohi):
                lo, hi = lohi
                mid = (lo + hi) // 2
                cnt = jnp.sum((eq & (idxw < mid)).astype(jnp.int32))
                take = cnt >= need
                return (jnp.where(take, lo, mid + 1), jnp.where(take, mid, hi))

            _, i0 = lax.fori_loop(0, 17, i0_step,
                                  (jnp.int32(0), jnp.int32(N_PAD)))
            ti_ref[0] = T
            ti_ref[1] = i0

        @pl.when(i > 0)
        def _():
            T = ti_ref[0]
            i0 = ti_ref[1]
            kv = key_ref[...]
            idx = (i - 1) * RB + lax.broadcasted_iota(jnp.int32, (RB,), 0)
            sel = (kv > T) | ((kv == T) & (idx < i0))
            sv = jnp.where(sel, sc_ref[...], 0.0)
            s_ref[...] = sv
            a_ref[...] = sel.astype(jnp.float32)
            vals = sv[None, :] * ht_ref[...]
            bm = jnp.max(jnp.where(sel[None, :], vals, -jnp.inf), axis=1,
                         keepdims=True)
            bs = jnp.sum(vals, axis=1, keepdims=True)

            @pl.when(i == 1)
            def _():
                mx_ref[...] = jnp.full((32, 1), -jnp.inf, jnp.float32)
                sm_ref[...] = jnp.zeros((32, 1), jnp.float32)

            mx_ref[...] = jnp.maximum(mx_ref[...], bm)
            sm_ref[...] = sm_ref[...] + bs * (1.0 / k)

    blk = lambda i: (jnp.maximum(i - 1, 0),)
    return pl.pallas_call(
        body,
        grid=(nb + 1,),
        in_specs=[
            pl.BlockSpec((N_PAD // 128, 128), lambda i: (0, 0)),
            pl.BlockSpec((RB,), blk),
            pl.BlockSpec((RB,), blk),
            pl.BlockSpec((32, RB), lambda i: (0, jnp.maximum(i - 1, 0))),
        ],
        out_specs=[
            pl.BlockSpec((RB,), blk),
            pl.BlockSpec((RB,), blk),
            pl.BlockSpec((32, 1), lambda i: (0, 0)),
            pl.BlockSpec((32, 1), lambda i: (0, 0)),
        ],
        out_shape=[
            jax.ShapeDtypeStruct((N_PAD,), jnp.float32),
            jax.ShapeDtypeStruct((N_PAD,), jnp.float32),
            jax.ShapeDtypeStruct((32, 1), jnp.float32),
            jax.ShapeDtypeStruct((32, 1), jnp.float32),
        ],
        scratch_shapes=[pltpu.SMEM((2,), jnp.int32)],
    )(keyw, key, score, hT)


def _tc_mlp(ros, lin1_W, lin1_b, lin2_W, lin2_b):
    def body(r1_ref, r2_ref, r3_ref, r4_ref, r5_ref, r6_ref,
             w1_ref, b1_ref, w2_ref, b2_ref, o_ref):
        z1 = b1_ref[...][None, :]
        for j, r in enumerate((r1_ref, r2_ref, r3_ref, r4_ref, r5_ref, r6_ref)):
            seg = w1_ref[pl.ds(32 * j, 32), :]
            z1 = z1 + jnp.sum(r[...] * seg, axis=0, keepdims=True)
        z1 = jnp.maximum(z1, 0.0)
        z2 = jnp.maximum(jnp.dot(z1, w2_ref[...],
                                 preferred_element_type=jnp.float32)
                         + b2_ref[...][None, :], 0.0)
        o_ref[...] = jnp.concatenate(
            [jnp.exp(z2[:, 0:3]), jax.nn.sigmoid(z2[:, 3:10])], axis=1)

    return pl.pallas_call(
        body,
        out_shape=jax.ShapeDtypeStruct((1, 10), jnp.float32),
    )(*ros, lin1_W, lin1_b, lin2_W, lin2_b)


def kernel(x, edge_index, W1, b1, p1, W2, b2, p2, W3, b3, p3,
           lin1_W, lin1_b, lin2_W, lin2_b):
    src = edge_index[0].astype(jnp.int32)
    dst = edge_index[1].astype(jnp.int32)
    pad = jnp.arange(E_PAD - E, dtype=jnp.int32)
    srcp = jnp.concatenate([src, pad % N])
    dstp = jnp.concatenate([dst, N + pad % (N_PAD - N)])
    src2 = jnp.concatenate([srcp, srcp + N_PAD])

    za = jnp.zeros((ROWS_PW,), jnp.float32)
    zb = jnp.zeros((ROWS_PW, 16), jnp.float32)

    h = jnp.concatenate([x, jnp.zeros((N_PAD - N, x.shape[1]), jnp.float32)])
    a = jnp.concatenate([jnp.ones((N,), jnp.float32),
                         jnp.zeros((N_PAD - N,), jnp.float32)])
    s = a

    n_alive = N
    readouts = []
    for (W, b, p) in ((W1, b1, p1), (W2, b2, p2), (W3, b3, p3)):
        cdeg = _sc_deg(a, srcp, dstp, za).reshape(2, N_PAD)
        y2, dis, xw = _tc_mprep(cdeg, a, s, h, W)
        acc2 = _sc_gather_add(y2.reshape(2 * N_PAD, 16), src2, dstp, zb)
        acc2 = acc2.reshape(2, N_PAD, 16)
        h, hT, score, key = _tc_post(acc2, xw, dis, a, b, p)
        k = int(math.ceil(0.6 * n_alive))
        s, a, mx, sm = _tc_searchsel(key.reshape(N_PAD // 128, 128), key,
                                     score, hT, k)
        readouts.extend([mx, sm])
        n_alive = k

    return _tc_mlp(readouts, lin1_W, lin1_b, lin2_W, lin2_b)
